# baseline (speedup 1.0000x reference)
import jax
import jax.numpy as jnp
from jax.experimental import pallas as pl
from jax.experimental.pallas import tpu as pltpu
from jax.experimental.pallas import tpu_sc as plsc

N_NODES = 10000
N_EDGES = 320000
NUM_GRAPHS = 16
D = 128
DE = 16

NC = 2
NS = 16
LANES = 16
NW = NC * NS
EB = 128
NCHUNKS = N_EDGES // EB
ROWS_PT = 632
NROW_ACC = NS * ROWS_PT


def _leaky(v):
    return jnp.maximum(v, 0.2 * v)


def _epre1_body(ea_ref, w_ref, b_ref, o_ref):
    z = jnp.dot(ea_ref[...], w_ref[...], preferred_element_type=jnp.float32)
    o_ref[...] = z + b_ref[...]


def _epre1(edge_attr, w, b):
    BE = 2000
    return pl.pallas_call(
        _epre1_body,
        grid=(N_EDGES // BE,),
        in_specs=[
            pl.BlockSpec((BE, DE), lambda i: (i, 0)),
            pl.BlockSpec((DE, D), lambda i: (0, 0)),
            pl.BlockSpec((1, D), lambda i: (0, 0)),
        ],
        out_specs=pl.BlockSpec((BE, D), lambda i: (i, 0)),
        out_shape=jax.ShapeDtypeStruct((N_EDGES, D), jnp.float32),
    )(edge_attr, w, b)


def _epre2_body(ea_ref, w_ref, b_ref, o1_ref, o2_ref):
    z = jnp.dot(ea_ref[...], w_ref[...], preferred_element_type=jnp.float32)
    z = z + b_ref[...]
    o1_ref[...] = z[:, :D]
    o2_ref[...] = z[:, D:]


def _epre2(edge_attr, w_cat, b_cat):
    BE = 2000
    out = jax.ShapeDtypeStruct((N_EDGES, D), jnp.float32)
    return pl.pallas_call(
        _epre2_body,
        grid=(N_EDGES // BE,),
        in_specs=[
            pl.BlockSpec((BE, DE), lambda i: (i, 0)),
            pl.BlockSpec((DE, 2 * D), lambda i: (0, 0)),
            pl.BlockSpec((1, 2 * D), lambda i: (0, 0)),
        ],
        out_specs=[
            pl.BlockSpec((BE, D), lambda i: (i, 0)),
            pl.BlockSpec((BE, D), lambda i: (i, 0)),
        ],
        out_shape=[out, out],
    )(edge_attr, w_cat, b_cat)


def _apre_body(h_ref, w_ref, o_ref):
    o_ref[...] = jnp.dot(h_ref[...], w_ref[...],
                         preferred_element_type=jnp.float32)


def _apre(h, w_top):
    return pl.pallas_call(
        _apre_body,
        out_shape=jax.ShapeDtypeStruct((N_NODES, D), jnp.float32),
    )(h, w_top)


def _update_body(p_ref, c_ref, h_ref, wt_ref, wb_ref, bu_ref, wn_ref,
                 oh_ref, oa_ref):
    cnt = c_ref[0, :, 0:1] + c_ref[1, :, 0:1]
    aggr = (p_ref[0] + p_ref[1]) / jnp.maximum(cnt, 1.0)
    z = jnp.dot(aggr, wt_ref[...], preferred_element_type=jnp.float32)
    z = z + jnp.dot(h_ref[...], wb_ref[...],
                    preferred_element_type=jnp.float32)
    hn = _leaky(z + bu_ref[...])
    oh_ref[...] = hn
    oa_ref[...] = jnp.dot(hn, wn_ref[...], preferred_element_type=jnp.float32)


def _update(partial, counts, h, wu_top, wu_bot, bu, w_next_top):
    return pl.pallas_call(
        _update_body,
        out_shape=[
            jax.ShapeDtypeStruct((N_NODES, D), jnp.float32),
            jax.ShapeDtypeStruct((N_NODES, D), jnp.float32),
        ],
    )(partial, counts, h, wu_top, wu_bot, bu, w_next_top)


def _finale_body(p_ref, c_ref, h_ref, wt_ref, wb_ref, bu_ref, b_ref,
                 w1_ref, b1_ref, w2_ref, b2_ref, w3_ref, b3_ref, o_ref):
    cnt = c_ref[0, :, 0:1] + c_ref[1, :, 0:1]
    aggr = (p_ref[0] + p_ref[1]) / jnp.maximum(cnt, 1.0)
    z = jnp.dot(aggr, wt_ref[...], preferred_element_type=jnp.float32)
    z = z + jnp.dot(h_ref[...], wb_ref[...],
                    preferred_element_type=jnp.float32)
    h3 = _leaky(z + bu_ref[...])
    gids = jax.lax.broadcasted_iota(jnp.int32, (NUM_GRAPHS, N_NODES), 0)
    mask = (b_ref[...] == gids).astype(jnp.float32)
    sums = jnp.dot(mask, h3, preferred_element_type=jnp.float32)
    gcnt = jnp.sum(mask, axis=1, keepdims=True)
    g = sums / jnp.maximum(gcnt, 1.0)
    g = _leaky(jnp.dot(g, w1_ref[...], preferred_element_type=jnp.float32)
               + b1_ref[...])
    g = _leaky(jnp.dot(g, w2_ref[...], preferred_element_type=jnp.float32)
               + b2_ref[...])
    o_ref[...] = (jnp.dot(g, w3_ref[...], preferred_element_type=jnp.float32)
                  + b3_ref[...])


def _finale(partial, counts, h, wu_top, wu_bot, bu, batch2d,
            w1, b1, w2, b2, w3, b3):
    return pl.pallas_call(
        _finale_body,
        out_shape=jax.ShapeDtypeStruct((NUM_GRAPHS, 1), jnp.float32),
    )(partial, counts, h, wu_top, wu_bot, bu, batch2d, w1, b1, w2, b2, w3, b3)


ECH = 64
IBK = 4
BLK = IBK * ECH
NBLK = N_EDGES // BLK
NITER = (NBLK + 2 * NW - 1) // (2 * NW)


def _sc_edge(apre, epre, src3d, dst3d, zeros_nd):
    mesh = plsc.VectorSubcoreMesh(core_axis_name="c", subcore_axis_name="s")

    def body(apre_hbm, epre_hbm, src_hbm, dst_hbm, z_hbm, out_hbm,
             gbuf0, ebuf0, seme0, semg0,
             gbuf1, ebuf1, seme1, semg1,
             sblk0, dblk0, sblk1, dblk1, acc):
        cid = jax.lax.axis_index("c")
        sid = jax.lax.axis_index("s")
        wid = sid * NC + cid
        row0 = pl.multiple_of(sid * ROWS_PT, 8)

        data = ((gbuf0, ebuf0, seme0, semg0), (gbuf1, ebuf1, seme1, semg1))
        iblk = ((sblk0, dblk0), (sblk1, dblk1))

        def load_iblk(b, ib):
            sblk, dblk = iblk[ib]
            pltpu.sync_copy(src_hbm.at[b], sblk)
            pltpu.sync_copy(dst_hbm.at[b], dblk)

        def start_chunk(b, c, ib, d):
            sblk, _ = iblk[ib]
            gbuf, ebuf, seme, semg = data[d]
            off = pl.multiple_of(b * BLK + c * ECH, 8)
            pltpu.make_async_copy(epre_hbm.at[pl.ds(off, ECH)], ebuf,
                                  seme).start()
            pltpu.make_async_copy(apre_hbm.at[sblk.at[c]], gbuf, semg).start()

        def finish_chunk(c, ib, d):
            sblk, dblk = iblk[ib]
            gbuf, ebuf, seme, semg = data[d]
            pltpu.make_async_copy(epre_hbm.at[pl.ds(0, ECH)], ebuf,
                                  seme).wait()
            pltpu.make_async_copy(apre_hbm.at[sblk.at[c]], gbuf, semg).wait()

            @pl.loop(0, ECH, step=4, unroll=2)
            def _row(i):
                for rr in range(4):
                    for j in range(D // LANES):
                        sl = pl.ds(j * LANES, LANES)
                        m = gbuf[i + rr, sl] + ebuf[i + rr, sl]
                        gbuf[i + rr, sl] = jnp.maximum(m, 0.2 * m)

            pltpu.sync_copy(gbuf, acc.at[dblk.at[c]], add=True)

        pltpu.sync_copy(z_hbm.at[sid], acc.at[pl.ds(row0, ROWS_PT)])
        plsc.subcore_barrier()

        load_iblk(wid, 0)
        start_chunk(wid, 0, 0, 0)

        @pl.loop(0, NITER)
        def _iter(t):
            b0 = wid + (2 * t) * NW
            b1 = b0 + NW
            bnext = b0 + 2 * NW
            gb1 = b1 < NBLK
            gnext = bnext < NBLK

            @pl.when(gb1)
            def _():
                load_iblk(b1, 1)

            start_chunk(b0, 1, 0, 1)
            finish_chunk(0, 0, 0)
            start_chunk(b0, 2, 0, 0)
            finish_chunk(1, 0, 1)
            start_chunk(b0, 3, 0, 1)
            finish_chunk(2, 0, 0)

            @pl.when(gb1)
            def _():
                start_chunk(b1, 0, 1, 0)

            finish_chunk(3, 0, 1)

            @pl.when(gb1)
            def _():
                start_chunk(b1, 1, 1, 1)
                finish_chunk(0, 1, 0)

                @pl.when(gnext)
                def _():
                    load_iblk(bnext, 0)

                start_chunk(b1, 2, 1, 0)
                finish_chunk(1, 1, 1)
                start_chunk(b1, 3, 1, 1)
                finish_chunk(2, 1, 0)

                @pl.when(gnext)
                def _():
                    start_chunk(bnext, 0, 0, 0)

                finish_chunk(3, 1, 1)

        plsc.subcore_barrier()
        pltpu.sync_copy(acc.at[pl.ds(row0, ROWS_PT)], out_hbm.at[cid, sid])

    k = pl.kernel(
        body,
        out_type=jax.ShapeDtypeStruct((NC, NS, ROWS_PT, D), jnp.float32),
        mesh=mesh,
        scratch_types=[
            pltpu.VMEM((ECH, D), jnp.float32),
            pltpu.VMEM((ECH, D), jnp.float32),
            pltpu.SemaphoreType.DMA,
            pltpu.SemaphoreType.DMA,
            pltpu.VMEM((ECH, D), jnp.float32),
            pltpu.VMEM((ECH, D), jnp.float32),
            pltpu.SemaphoreType.DMA,
            pltpu.SemaphoreType.DMA,
            pltpu.VMEM((IBK, ECH), jnp.int32),
            pltpu.VMEM((IBK, ECH), jnp.int32),
            pltpu.VMEM((IBK, ECH), jnp.int32),
            pltpu.VMEM((IBK, ECH), jnp.int32),
            pltpu.VMEM_SHARED((NROW_ACC, D), jnp.float32),
        ],
    )
    res = k(apre, epre, src3d, dst3d, zeros_nd)
    return res.reshape(NC, NROW_ACC, D)[:, :N_NODES]


def _sc_counts(dst, zeros_nd, ones_ebd):
    mesh = plsc.VectorSubcoreMesh(core_axis_name="c", subcore_axis_name="s")

    def body(dst_hbm, z_hbm, ones_hbm, cout_hbm, didx0, didx1, semi0, semi1,
             didxt, obuf, cacc):
        cid = jax.lax.axis_index("c")
        sid = jax.lax.axis_index("s")
        wid = sid * NC + cid
        row0 = pl.multiple_of(sid * ROWS_PT, 8)
        base = pl.multiple_of(wid * (N_EDGES // NW), 8)
        nch = (N_EDGES // NW) // EB

        ibufs = ((didx0, semi0), (didx1, semi1))

        def start_idx(ch, b):
            didx, semi = ibufs[b]
            off = pl.multiple_of(base + ch * EB, 8)
            pltpu.make_async_copy(dst_hbm.at[pl.ds(off, EB)], didx,
                                  semi).start()

        def finish(ch, b):
            didx, semi = ibufs[b]
            off = pl.multiple_of(base + ch * EB, 8)
            pltpu.make_async_copy(dst_hbm.at[pl.ds(off, EB)], didx,
                                  semi).wait()
            pltpu.sync_copy(obuf, cacc.at[didx], add=True)

        pltpu.sync_copy(z_hbm.at[sid], cacc.at[pl.ds(row0, ROWS_PT)])
        pltpu.sync_copy(ones_hbm, obuf)
        plsc.subcore_barrier()

        start_idx(0, 0)
        start_idx(1, 1)

        @pl.loop(0, nch // 2)
        def _pair(j):
            finish(2 * j, 0)

            @pl.when(j < nch // 2 - 1)
            def _():
                start_idx(2 * j + 2, 0)

            finish(2 * j + 1, 1)

            @pl.when(j < nch // 2 - 1)
            def _():
                start_idx(2 * j + 3, 1)

        offt = pl.multiple_of(base + nch * EB, 8)
        pltpu.sync_copy(dst_hbm.at[pl.ds(offt, 16)], didxt)
        pltpu.sync_copy(obuf.at[pl.ds(0, 16)], cacc.at[didxt], add=True)

        plsc.subcore_barrier()
        pltpu.sync_copy(cacc.at[pl.ds(row0, ROWS_PT)], cout_hbm.at[cid, sid])

    k = pl.kernel(
        body,
        out_type=jax.ShapeDtypeStruct((NC, NS, ROWS_PT, D), jnp.float32),
        mesh=mesh,
        scratch_types=[
            pltpu.VMEM((EB,), jnp.int32),
            pltpu.VMEM((EB,), jnp.int32),
            pltpu.SemaphoreType.DMA,
            pltpu.SemaphoreType.DMA,
            pltpu.VMEM((16,), jnp.int32),
            pltpu.VMEM((EB, D), jnp.float32),
            pltpu.VMEM_SHARED((NROW_ACC, D), jnp.float32),
        ],
    )
    res = k(dst, zeros_nd, ones_ebd)
    return res.reshape(NC, NROW_ACC, D)[:, :N_NODES, :LANES]


def kernel(x, edge_index, edge_attr, batch,
           Wm0, bm0, Wu0, bu0, Wm1, bm1, Wu1, bu1, Wm2, bm2, Wu2, bu2,
           W1, b1, W2, b2, W3, b3):
    src = edge_index[0]
    dst = edge_index[1]
    src3d = src.reshape(NBLK, IBK, ECH)
    dst3d = dst.reshape(NBLK, IBK, ECH)
    zeros_nd = jnp.zeros((NS, ROWS_PT, D), jnp.float32)
    ones_ebd = jnp.ones((EB, D), jnp.float32)

    e0 = _epre1(edge_attr, Wm0[D:], bm0.reshape(1, D))
    cnts = _sc_counts(dst, zeros_nd, ones_ebd)
    apre0 = _apre(x, Wm0[:D])

    e1 = _epre1(edge_attr, Wm1[D:], bm1.reshape(1, D))
    e2 = _epre1(edge_attr, Wm2[D:], bm2.reshape(1, D))

    part0 = _sc_edge(apre0, e0, src3d, dst3d, zeros_nd)
    h1, apre1 = _update(part0, cnts, x, Wu0[:D], Wu0[D:],
                        bu0.reshape(1, D), Wm1[:D])
    part1 = _sc_edge(apre1, e1, src3d, dst3d, zeros_nd)
    h2, apre2 = _update(part1, cnts, h1, Wu1[:D], Wu1[D:],
                        bu1.reshape(1, D), Wm2[:D])
    part2 = _sc_edge(apre2, e2, src3d, dst3d, zeros_nd)

    return _finale(part2, cnts, h2, Wu2[:D], Wu2[D:], bu2.reshape(1, D),
                   batch.reshape(1, N_NODES), W1, b1.reshape(1, D),
                   W2, b2.reshape(1, 64), W3, b3.reshape(1, 1))


# speedup vs baseline: 1.9521x; 1.9521x over previous
import jax
import jax.numpy as jnp
from jax.experimental import pallas as pl
from jax.experimental.pallas import tpu as pltpu
from jax.experimental.pallas import tpu_sc as plsc

N_NODES = 10000
N_EDGES = 320000
NUM_GRAPHS = 16
D = 128
DE = 16

NC = 2
NS = 16
LANES = 16
NW = NC * NS
EB = 128
NCHUNKS = N_EDGES // EB
ROWS_PT = 632
NROW_ACC = NS * ROWS_PT


def _leaky(v):
    return jnp.maximum(v, 0.2 * v)


def _epre1_body(ea_ref, w_ref, b_ref, o_ref):
    z = jnp.dot(ea_ref[...], w_ref[...], preferred_element_type=jnp.float32)
    o_ref[...] = z + b_ref[...]


def _epre1(edge_attr, w, b):
    BE = 2000
    return pl.pallas_call(
        _epre1_body,
        grid=(N_EDGES // BE,),
        in_specs=[
            pl.BlockSpec((BE, DE), lambda i: (i, 0)),
            pl.BlockSpec((DE, D), lambda i: (0, 0)),
            pl.BlockSpec((1, D), lambda i: (0, 0)),
        ],
        out_specs=pl.BlockSpec((BE, D), lambda i: (i, 0)),
        out_shape=jax.ShapeDtypeStruct((N_EDGES, D), jnp.float32),
    )(edge_attr, w, b)


def _epre2_body(ea_ref, w_ref, b_ref, o1_ref, o2_ref):
    z = jnp.dot(ea_ref[...], w_ref[...], preferred_element_type=jnp.float32)
    z = z + b_ref[...]
    o1_ref[...] = z[:, :D]
    o2_ref[...] = z[:, D:]


def _epre2(edge_attr, w_cat, b_cat):
    BE = 2000
    out = jax.ShapeDtypeStruct((N_EDGES, D), jnp.float32)
    return pl.pallas_call(
        _epre2_body,
        grid=(N_EDGES // BE,),
        in_specs=[
            pl.BlockSpec((BE, DE), lambda i: (i, 0)),
            pl.BlockSpec((DE, 2 * D), lambda i: (0, 0)),
            pl.BlockSpec((1, 2 * D), lambda i: (0, 0)),
        ],
        out_specs=[
            pl.BlockSpec((BE, D), lambda i: (i, 0)),
            pl.BlockSpec((BE, D), lambda i: (i, 0)),
        ],
        out_shape=[out, out],
    )(edge_attr, w_cat, b_cat)


def _apre_body(h_ref, w_ref, o_ref):
    o_ref[...] = jnp.dot(h_ref[...], w_ref[...],
                         preferred_element_type=jnp.float32)


def _apre(h, w_top):
    return pl.pallas_call(
        _apre_body,
        out_shape=jax.ShapeDtypeStruct((N_NODES, D), jnp.float32),
    )(h, w_top)


def _update_body(p_ref, c_ref, h_ref, wt_ref, wb_ref, bu_ref, wn_ref,
                 oh_ref, oa_ref):
    cnt = c_ref[0, :, 0:1] + c_ref[1, :, 0:1]
    aggr = (p_ref[0] + p_ref[1]) / jnp.maximum(cnt, 1.0)
    z = jnp.dot(aggr, wt_ref[...], preferred_element_type=jnp.float32)
    z = z + jnp.dot(h_ref[...], wb_ref[...],
                    preferred_element_type=jnp.float32)
    hn = _leaky(z + bu_ref[...])
    oh_ref[...] = hn
    oa_ref[...] = jnp.dot(hn, wn_ref[...], preferred_element_type=jnp.float32)


def _update(partial, counts, h, wu_top, wu_bot, bu, w_next_top):
    return pl.pallas_call(
        _update_body,
        out_shape=[
            jax.ShapeDtypeStruct((N_NODES, D), jnp.float32),
            jax.ShapeDtypeStruct((N_NODES, D), jnp.float32),
        ],
    )(partial, counts, h, wu_top, wu_bot, bu, w_next_top)


def _finale_body(p_ref, c_ref, h_ref, wt_ref, wb_ref, bu_ref, b_ref,
                 w1_ref, b1_ref, w2_ref, b2_ref, w3_ref, b3_ref, o_ref):
    cnt = c_ref[0, :, 0:1] + c_ref[1, :, 0:1]
    aggr = (p_ref[0] + p_ref[1]) / jnp.maximum(cnt, 1.0)
    z = jnp.dot(aggr, wt_ref[...], preferred_element_type=jnp.float32)
    z = z + jnp.dot(h_ref[...], wb_ref[...],
                    preferred_element_type=jnp.float32)
    h3 = _leaky(z + bu_ref[...])
    gids = jax.lax.broadcasted_iota(jnp.int32, (NUM_GRAPHS, N_NODES), 0)
    mask = (b_ref[...] == gids).astype(jnp.float32)
    sums = jnp.dot(mask, h3, preferred_element_type=jnp.float32)
    gcnt = jnp.sum(mask, axis=1, keepdims=True)
    g = sums / jnp.maximum(gcnt, 1.0)
    g = _leaky(jnp.dot(g, w1_ref[...], preferred_element_type=jnp.float32)
               + b1_ref[...])
    g = _leaky(jnp.dot(g, w2_ref[...], preferred_element_type=jnp.float32)
               + b2_ref[...])
    o_ref[...] = (jnp.dot(g, w3_ref[...], preferred_element_type=jnp.float32)
                  + b3_ref[...])


def _finale(partial, counts, h, wu_top, wu_bot, bu, batch2d,
            w1, b1, w2, b2, w3, b3):
    return pl.pallas_call(
        _finale_body,
        out_shape=jax.ShapeDtypeStruct((NUM_GRAPHS, 1), jnp.float32),
    )(partial, counts, h, wu_top, wu_bot, bu, batch2d, w1, b1, w2, b2, w3, b3)


ECH = 64
IBK = 4
BLK = IBK * ECH
NBLK = N_EDGES // BLK
NITER = (NBLK + 2 * NW - 1) // (2 * NW)


def _sc_edge(apre, epre, src3d, dst3d, zeros_nd):
    mesh = plsc.VectorSubcoreMesh(core_axis_name="c", subcore_axis_name="s")

    def body(apre_hbm, epre_hbm, src_hbm, dst_hbm, z_hbm, out_hbm,
             gbuf0, ebuf0, seme0, semg0,
             gbuf1, ebuf1, seme1, semg1,
             sblk0, dblk0, sblk1, dblk1, acc):
        cid = jax.lax.axis_index("c")
        sid = jax.lax.axis_index("s")
        wid = sid * NC + cid
        row0 = pl.multiple_of(sid * ROWS_PT, 8)

        data = ((gbuf0, ebuf0, seme0, semg0), (gbuf1, ebuf1, seme1, semg1))
        iblk = ((sblk0, dblk0), (sblk1, dblk1))

        def load_iblk(b, ib):
            sblk, dblk = iblk[ib]
            pltpu.sync_copy(src_hbm.at[b], sblk)
            pltpu.sync_copy(dst_hbm.at[b], dblk)

        def start_chunk(b, c, ib, d):
            sblk, _ = iblk[ib]
            gbuf, ebuf, seme, semg = data[d]
            off = pl.multiple_of(b * BLK + c * ECH, 8)
            pltpu.make_async_copy(epre_hbm.at[pl.ds(off, ECH)], ebuf,
                                  seme).start()
            pltpu.make_async_copy(apre_hbm.at[sblk.at[c]], gbuf, semg).start()

        def finish_chunk(c, ib, d):
            sblk, dblk = iblk[ib]
            gbuf, ebuf, seme, semg = data[d]
            pltpu.make_async_copy(epre_hbm.at[pl.ds(0, ECH)], ebuf,
                                  seme).wait()
            pltpu.make_async_copy(apre_hbm.at[sblk.at[c]], gbuf, semg).wait()

            @pl.loop(0, ECH, step=4)
            def _row(i):
                for rr in range(4):
                    for j in range(D // LANES):
                        sl = pl.ds(j * LANES, LANES)
                        m = gbuf[i + rr, sl] + ebuf[i + rr, sl]
                        gbuf[i + rr, sl] = jnp.maximum(m, 0.2 * m)

            pltpu.sync_copy(gbuf, acc.at[dblk.at[c]], add=True)

        pltpu.sync_copy(z_hbm.at[sid], acc.at[pl.ds(row0, ROWS_PT)])
        plsc.subcore_barrier()

        load_iblk(wid, 0)
        start_chunk(wid, 0, 0, 0)

        @pl.loop(0, NITER)
        def _iter(t):
            b0 = wid + (2 * t) * NW
            b1 = b0 + NW
            bnext = b0 + 2 * NW
            gb1 = b1 < NBLK
            gnext = bnext < NBLK

            @pl.when(gb1)
            def _():
                load_iblk(b1, 1)

            start_chunk(b0, 1, 0, 1)
            finish_chunk(0, 0, 0)
            start_chunk(b0, 2, 0, 0)
            finish_chunk(1, 0, 1)
            start_chunk(b0, 3, 0, 1)
            finish_chunk(2, 0, 0)

            @pl.when(gb1)
            def _():
                start_chunk(b1, 0, 1, 0)

            finish_chunk(3, 0, 1)

            @pl.when(gb1)
            def _():
                start_chunk(b1, 1, 1, 1)
                finish_chunk(0, 1, 0)

                @pl.when(gnext)
                def _():
                    load_iblk(bnext, 0)

                start_chunk(b1, 2, 1, 0)
                finish_chunk(1, 1, 1)
                start_chunk(b1, 3, 1, 1)
                finish_chunk(2, 1, 0)

                @pl.when(gnext)
                def _():
                    start_chunk(bnext, 0, 0, 0)

                finish_chunk(3, 1, 1)

        plsc.subcore_barrier()
        pltpu.sync_copy(acc.at[pl.ds(row0, ROWS_PT)], out_hbm.at[cid, sid])

    k = pl.kernel(
        body,
        out_type=jax.ShapeDtypeStruct((NC, NS, ROWS_PT, D), jnp.float32),
        mesh=mesh,
        scratch_types=[
            pltpu.VMEM((ECH, D), jnp.float32),
            pltpu.VMEM((ECH, D), jnp.float32),
            pltpu.SemaphoreType.DMA,
            pltpu.SemaphoreType.DMA,
            pltpu.VMEM((ECH, D), jnp.float32),
            pltpu.VMEM((ECH, D), jnp.float32),
            pltpu.SemaphoreType.DMA,
            pltpu.SemaphoreType.DMA,
            pltpu.VMEM((IBK, ECH), jnp.int32),
            pltpu.VMEM((IBK, ECH), jnp.int32),
            pltpu.VMEM((IBK, ECH), jnp.int32),
            pltpu.VMEM((IBK, ECH), jnp.int32),
            pltpu.VMEM_SHARED((NROW_ACC, D), jnp.float32),
        ],
    )
    res = k(apre, epre, src3d, dst3d, zeros_nd)
    return res.reshape(NC, NROW_ACC, D)[:, :N_NODES]


def _sc_counts(dst, zeros_nd, ones_ebd):
    mesh = plsc.VectorSubcoreMesh(core_axis_name="c", subcore_axis_name="s")

    def body(dst_hbm, z_hbm, ones_hbm, cout_hbm, didx0, didx1, semi0, semi1,
             didxt, obuf, cacc):
        cid = jax.lax.axis_index("c")
        sid = jax.lax.axis_index("s")
        wid = sid * NC + cid
        row0 = pl.multiple_of(sid * ROWS_PT, 8)
        base = pl.multiple_of(wid * (N_EDGES // NW), 8)
        nch = (N_EDGES // NW) // EB

        ibufs = ((didx0, semi0), (didx1, semi1))

        def start_idx(ch, b):
            didx, semi = ibufs[b]
            off = pl.multiple_of(base + ch * EB, 8)
            pltpu.make_async_copy(dst_hbm.at[pl.ds(off, EB)], didx,
                                  semi).start()

        def finish(ch, b):
            didx, semi = ibufs[b]
            off = pl.multiple_of(base + ch * EB, 8)
            pltpu.make_async_copy(dst_hbm.at[pl.ds(off, EB)], didx,
                                  semi).wait()
            pltpu.sync_copy(obuf, cacc.at[didx], add=True)

        pltpu.sync_copy(z_hbm.at[sid], cacc.at[pl.ds(row0, ROWS_PT)])
        pltpu.sync_copy(ones_hbm, obuf)
        plsc.subcore_barrier()

        start_idx(0, 0)
        start_idx(1, 1)

        @pl.loop(0, nch // 2)
        def _pair(j):
            finish(2 * j, 0)

            @pl.when(j < nch // 2 - 1)
            def _():
                start_idx(2 * j + 2, 0)

            finish(2 * j + 1, 1)

            @pl.when(j < nch // 2 - 1)
            def _():
                start_idx(2 * j + 3, 1)

        offt = pl.multiple_of(base + nch * EB, 8)
        pltpu.sync_copy(dst_hbm.at[pl.ds(offt, 16)], didxt)
        pltpu.sync_copy(obuf.at[pl.ds(0, 16)], cacc.at[didxt], add=True)

        plsc.subcore_barrier()
        pltpu.sync_copy(cacc.at[pl.ds(row0, ROWS_PT)], cout_hbm.at[cid, sid])

    k = pl.kernel(
        body,
        out_type=jax.ShapeDtypeStruct((NC, NS, ROWS_PT, D), jnp.float32),
        mesh=mesh,
        scratch_types=[
            pltpu.VMEM((EB,), jnp.int32),
            pltpu.VMEM((EB,), jnp.int32),
            pltpu.SemaphoreType.DMA,
            pltpu.SemaphoreType.DMA,
            pltpu.VMEM((16,), jnp.int32),
            pltpu.VMEM((EB, D), jnp.float32),
            pltpu.VMEM_SHARED((NROW_ACC, D), jnp.float32),
        ],
    )
    res = k(dst, zeros_nd, ones_ebd)
    return res.reshape(NC, NROW_ACC, D)[:, :N_NODES, :LANES]


def kernel(x, edge_index, edge_attr, batch,
           Wm0, bm0, Wu0, bu0, Wm1, bm1, Wu1, bu1, Wm2, bm2, Wu2, bu2,
           W1, b1, W2, b2, W3, b3):
    src = edge_index[0]
    dst = edge_index[1]
    src3d = src.reshape(NBLK, IBK, ECH)
    dst3d = dst.reshape(NBLK, IBK, ECH)
    zeros_nd = jnp.zeros((NS, ROWS_PT, D), jnp.float32)
    ones_ebd = jnp.ones((EB, D), jnp.float32)

    e0 = _epre1(edge_attr, Wm0[D:], bm0.reshape(1, D))
    cnts = _sc_counts(dst, zeros_nd, ones_ebd)
    apre0 = _apre(x, Wm0[:D])

    e1 = _epre1(edge_attr, Wm1[D:], bm1.reshape(1, D))
    e2 = _epre1(edge_attr, Wm2[D:], bm2.reshape(1, D))

    part0 = _sc_edge(apre0, e0, src3d, dst3d, zeros_nd)
    h1, apre1 = _update(part0, cnts, x, Wu0[:D], Wu0[D:],
                        bu0.reshape(1, D), Wm1[:D])
    part1 = _sc_edge(apre1, e1, src3d, dst3d, zeros_nd)
    h2, apre2 = _update(part1, cnts, h1, Wu1[:D], Wu1[D:],
                        bu1.reshape(1, D), Wm2[:D])
    part2 = _sc_edge(apre2, e2, src3d, dst3d, zeros_nd)

    return _finale(part2, cnts, h2, Wu2[:D], Wu2[D:], bu2.reshape(1, D),
                   batch.reshape(1, N_NODES), W1, b1.reshape(1, D),
                   W2, b2.reshape(1, 64), W3, b3.reshape(1, 1))


# speedup vs baseline: 1.9544x; 1.0012x over previous
import jax
import jax.numpy as jnp
from jax.experimental import pallas as pl
from jax.experimental.pallas import tpu as pltpu
from jax.experimental.pallas import tpu_sc as plsc

N_NODES = 10000
N_EDGES = 320000
NUM_GRAPHS = 16
D = 128
DE = 16

NC = 2
NS = 16
LANES = 16
NW = NC * NS
EB = 128
NCHUNKS = N_EDGES // EB
ROWS_PT = 632
NROW_ACC = NS * ROWS_PT


def _leaky(v):
    return jnp.maximum(v, 0.2 * v)


def _epre1_body(ea_ref, w_ref, b_ref, o_ref):
    z = jnp.dot(ea_ref[...], w_ref[...], preferred_element_type=jnp.float32)
    o_ref[...] = z + b_ref[...]


def _epre1(edge_attr, w, b):
    BE = 2000
    return pl.pallas_call(
        _epre1_body,
        grid=(N_EDGES // BE,),
        in_specs=[
            pl.BlockSpec((BE, DE), lambda i: (i, 0)),
            pl.BlockSpec((DE, D), lambda i: (0, 0)),
            pl.BlockSpec((1, D), lambda i: (0, 0)),
        ],
        out_specs=pl.BlockSpec((BE, D), lambda i: (i, 0)),
        out_shape=jax.ShapeDtypeStruct((N_EDGES, D), jnp.float32),
    )(edge_attr, w, b)


def _epre2_body(ea_ref, w_ref, b_ref, o1_ref, o2_ref):
    z = jnp.dot(ea_ref[...], w_ref[...], preferred_element_type=jnp.float32)
    z = z + b_ref[...]
    o1_ref[...] = z[:, :D]
    o2_ref[...] = z[:, D:]


def _epre2(edge_attr, w_cat, b_cat):
    BE = 2000
    out = jax.ShapeDtypeStruct((N_EDGES, D), jnp.float32)
    return pl.pallas_call(
        _epre2_body,
        grid=(N_EDGES // BE,),
        in_specs=[
            pl.BlockSpec((BE, DE), lambda i: (i, 0)),
            pl.BlockSpec((DE, 2 * D), lambda i: (0, 0)),
            pl.BlockSpec((1, 2 * D), lambda i: (0, 0)),
        ],
        out_specs=[
            pl.BlockSpec((BE, D), lambda i: (i, 0)),
            pl.BlockSpec((BE, D), lambda i: (i, 0)),
        ],
        out_shape=[out, out],
    )(edge_attr, w_cat, b_cat)


def _apre_body(h_ref, w_ref, o_ref):
    o_ref[...] = jnp.dot(h_ref[...], w_ref[...],
                         preferred_element_type=jnp.float32)


def _apre(h, w_top):
    return pl.pallas_call(
        _apre_body,
        out_shape=jax.ShapeDtypeStruct((N_NODES, D), jnp.float32),
    )(h, w_top)


def _update_body(p_ref, c_ref, h_ref, wt_ref, wb_ref, bu_ref, wn_ref,
                 oh_ref, oa_ref):
    cnt = c_ref[0, :, 0:1] + c_ref[1, :, 0:1]
    aggr = (p_ref[0] + p_ref[1]) / jnp.maximum(cnt, 1.0)
    z = jnp.dot(aggr, wt_ref[...], preferred_element_type=jnp.float32)
    z = z + jnp.dot(h_ref[...], wb_ref[...],
                    preferred_element_type=jnp.float32)
    hn = _leaky(z + bu_ref[...])
    oh_ref[...] = hn
    oa_ref[...] = jnp.dot(hn, wn_ref[...], preferred_element_type=jnp.float32)


def _update(partial, counts, h, wu_top, wu_bot, bu, w_next_top):
    return pl.pallas_call(
        _update_body,
        out_shape=[
            jax.ShapeDtypeStruct((N_NODES, D), jnp.float32),
            jax.ShapeDtypeStruct((N_NODES, D), jnp.float32),
        ],
    )(partial, counts, h, wu_top, wu_bot, bu, w_next_top)


def _finale_body(p_ref, c_ref, h_ref, wt_ref, wb_ref, bu_ref, b_ref,
                 w1_ref, b1_ref, w2_ref, b2_ref, w3_ref, b3_ref, o_ref):
    cnt = c_ref[0, :, 0:1] + c_ref[1, :, 0:1]
    aggr = (p_ref[0] + p_ref[1]) / jnp.maximum(cnt, 1.0)
    z = jnp.dot(aggr, wt_ref[...], preferred_element_type=jnp.float32)
    z = z + jnp.dot(h_ref[...], wb_ref[...],
                    preferred_element_type=jnp.float32)
    h3 = _leaky(z + bu_ref[...])
    gids = jax.lax.broadcasted_iota(jnp.int32, (NUM_GRAPHS, N_NODES), 0)
    mask = (b_ref[...] == gids).astype(jnp.float32)
    sums = jnp.dot(mask, h3, preferred_element_type=jnp.float32)
    gcnt = jnp.sum(mask, axis=1, keepdims=True)
    g = sums / jnp.maximum(gcnt, 1.0)
    g = _leaky(jnp.dot(g, w1_ref[...], preferred_element_type=jnp.float32)
               + b1_ref[...])
    g = _leaky(jnp.dot(g, w2_ref[...], preferred_element_type=jnp.float32)
               + b2_ref[...])
    o_ref[...] = (jnp.dot(g, w3_ref[...], preferred_element_type=jnp.float32)
                  + b3_ref[...])


def _finale(partial, counts, h, wu_top, wu_bot, bu, batch2d,
            w1, b1, w2, b2, w3, b3):
    return pl.pallas_call(
        _finale_body,
        out_shape=jax.ShapeDtypeStruct((NUM_GRAPHS, 1), jnp.float32),
    )(partial, counts, h, wu_top, wu_bot, bu, batch2d, w1, b1, w2, b2, w3, b3)


ECH = 64
IBK = 4
BLK = IBK * ECH
NBLK = N_EDGES // BLK
NITER = (NBLK + 2 * NW - 1) // (2 * NW)


def _sc_edge(apre, epre, src3d, dst3d, zeros_nd):
    mesh = plsc.VectorSubcoreMesh(core_axis_name="c", subcore_axis_name="s")

    def body(apre_hbm, epre_hbm, src_hbm, dst_hbm, z_hbm, out_hbm,
             gbuf0, ebuf0, seme0, semg0,
             gbuf1, ebuf1, seme1, semg1,
             sblk0, dblk0, sblk1, dblk1, acc):
        cid = jax.lax.axis_index("c")
        sid = jax.lax.axis_index("s")
        wid = sid * NC + cid
        row0 = pl.multiple_of(sid * ROWS_PT, 8)

        data = ((gbuf0, ebuf0, seme0, semg0), (gbuf1, ebuf1, seme1, semg1))
        iblk = ((sblk0, dblk0), (sblk1, dblk1))

        def load_iblk(b, ib):
            sblk, dblk = iblk[ib]
            pltpu.sync_copy(src_hbm.at[b], sblk)
            pltpu.sync_copy(dst_hbm.at[b], dblk)

        def start_chunk(b, c, ib, d):
            sblk, _ = iblk[ib]
            gbuf, ebuf, seme, semg = data[d]
            off = pl.multiple_of(b * BLK + c * ECH, 8)
            pltpu.make_async_copy(epre_hbm.at[pl.ds(off, ECH)], ebuf,
                                  seme).start()
            pltpu.make_async_copy(apre_hbm.at[sblk.at[c]], gbuf, semg).start()

        def finish_chunk(c, ib, d):
            sblk, dblk = iblk[ib]
            gbuf, ebuf, seme, semg = data[d]
            pltpu.make_async_copy(epre_hbm.at[pl.ds(0, ECH)], ebuf,
                                  seme).wait()
            pltpu.make_async_copy(apre_hbm.at[sblk.at[c]], gbuf, semg).wait()

            @pl.loop(0, ECH, step=4)
            def _row(i):
                for rr in range(4):
                    for j in range(D // LANES):
                        sl = pl.ds(j * LANES, LANES)
                        m = gbuf[i + rr, sl] + ebuf[i + rr, sl]
                        gbuf[i + rr, sl] = jnp.maximum(m, 0.2 * m)

            pltpu.sync_copy(gbuf, acc.at[dblk.at[c]], add=True)

        pltpu.sync_copy(z_hbm.at[sid], acc.at[pl.ds(row0, ROWS_PT)])
        plsc.subcore_barrier()

        load_iblk(wid, 0)
        start_chunk(wid, 0, 0, 0)

        @pl.loop(0, NITER)
        def _iter(t):
            b0 = wid + (2 * t) * NW
            b1 = b0 + NW
            bnext = b0 + 2 * NW
            gb1 = b1 < NBLK
            gnext = bnext < NBLK

            @pl.when(gb1)
            def _():
                load_iblk(b1, 1)

            start_chunk(b0, 1, 0, 1)
            finish_chunk(0, 0, 0)
            start_chunk(b0, 2, 0, 0)
            finish_chunk(1, 0, 1)
            start_chunk(b0, 3, 0, 1)
            finish_chunk(2, 0, 0)

            @pl.when(gb1)
            def _():
                start_chunk(b1, 0, 1, 0)

            finish_chunk(3, 0, 1)

            @pl.when(gb1)
            def _():
                start_chunk(b1, 1, 1, 1)
                finish_chunk(0, 1, 0)

                @pl.when(gnext)
                def _():
                    load_iblk(bnext, 0)

                start_chunk(b1, 2, 1, 0)
                finish_chunk(1, 1, 1)
                start_chunk(b1, 3, 1, 1)
                finish_chunk(2, 1, 0)

                @pl.when(gnext)
                def _():
                    start_chunk(bnext, 0, 0, 0)

                finish_chunk(3, 1, 1)

        plsc.subcore_barrier()
        pltpu.sync_copy(acc.at[pl.ds(row0, ROWS_PT)], out_hbm.at[cid, sid])

    k = pl.kernel(
        body,
        out_type=jax.ShapeDtypeStruct((NC, NS, ROWS_PT, D), jnp.float32),
        mesh=mesh,
        scratch_types=[
            pltpu.VMEM((ECH, D), jnp.float32),
            pltpu.VMEM((ECH, D), jnp.float32),
            pltpu.SemaphoreType.DMA,
            pltpu.SemaphoreType.DMA,
            pltpu.VMEM((ECH, D), jnp.float32),
            pltpu.VMEM((ECH, D), jnp.float32),
            pltpu.SemaphoreType.DMA,
            pltpu.SemaphoreType.DMA,
            pltpu.VMEM((IBK, ECH), jnp.int32),
            pltpu.VMEM((IBK, ECH), jnp.int32),
            pltpu.VMEM((IBK, ECH), jnp.int32),
            pltpu.VMEM((IBK, ECH), jnp.int32),
            pltpu.VMEM_SHARED((NROW_ACC, D), jnp.float32),
        ],
    )
    res = k(apre, epre, src3d, dst3d, zeros_nd)
    return res.reshape(NC, NROW_ACC, D)[:, :N_NODES]


def _sc_counts(dst, zeros_nd, ones_ebd):
    mesh = plsc.VectorSubcoreMesh(core_axis_name="c", subcore_axis_name="s")

    def body(dst_hbm, z_hbm, ones_hbm, cout_hbm, didx0, didx1, semi0, semi1,
             didxt, obuf, cacc):
        cid = jax.lax.axis_index("c")
        sid = jax.lax.axis_index("s")
        wid = sid * NC + cid
        row0 = pl.multiple_of(sid * ROWS_PT, 8)
        base = pl.multiple_of(wid * (N_EDGES // NW), 8)
        nch = (N_EDGES // NW) // EB

        ibufs = ((didx0, semi0), (didx1, semi1))

        def start_idx(ch, b):
            didx, semi = ibufs[b]
            off = pl.multiple_of(base + ch * EB, 8)
            pltpu.make_async_copy(dst_hbm.at[pl.ds(off, EB)], didx,
                                  semi).start()

        def finish(ch, b):
            didx, semi = ibufs[b]
            off = pl.multiple_of(base + ch * EB, 8)
            pltpu.make_async_copy(dst_hbm.at[pl.ds(off, EB)], didx,
                                  semi).wait()
            pltpu.sync_copy(obuf, cacc.at[didx], add=True)

        pltpu.sync_copy(z_hbm.at[sid], cacc.at[pl.ds(row0, ROWS_PT)])
        pltpu.sync_copy(ones_hbm, obuf)
        plsc.subcore_barrier()

        start_idx(0, 0)
        start_idx(1, 1)

        @pl.loop(0, nch // 2)
        def _pair(j):
            finish(2 * j, 0)

            @pl.when(j < nch // 2 - 1)
            def _():
                start_idx(2 * j + 2, 0)

            finish(2 * j + 1, 1)

            @pl.when(j < nch // 2 - 1)
            def _():
                start_idx(2 * j + 3, 1)

        offt = pl.multiple_of(base + nch * EB, 8)
        pltpu.sync_copy(dst_hbm.at[pl.ds(offt, 16)], didxt)
        pltpu.sync_copy(obuf.at[pl.ds(0, 16)], cacc.at[didxt], add=True)

        plsc.subcore_barrier()
        pltpu.sync_copy(cacc.at[pl.ds(row0, ROWS_PT)], cout_hbm.at[cid, sid])

    k = pl.kernel(
        body,
        out_type=jax.ShapeDtypeStruct((NC, NS, ROWS_PT, D), jnp.float32),
        mesh=mesh,
        scratch_types=[
            pltpu.VMEM((EB,), jnp.int32),
            pltpu.VMEM((EB,), jnp.int32),
            pltpu.SemaphoreType.DMA,
            pltpu.SemaphoreType.DMA,
            pltpu.VMEM((16,), jnp.int32),
            pltpu.VMEM((EB, D), jnp.float32),
            pltpu.VMEM_SHARED((NROW_ACC, D), jnp.float32),
        ],
    )
    res = k(dst, zeros_nd, ones_ebd)
    return res.reshape(NC, NROW_ACC, D)[:, :N_NODES, :LANES]


def kernel(x, edge_index, edge_attr, batch,
           Wm0, bm0, Wu0, bu0, Wm1, bm1, Wu1, bu1, Wm2, bm2, Wu2, bu2,
           W1, b1, W2, b2, W3, b3):
    src = edge_index[0]
    dst = edge_index[1]
    src3d = src.reshape(NBLK, IBK, ECH)
    dst3d = dst.reshape(NBLK, IBK, ECH)
    zeros_nd = jnp.zeros((NS, ROWS_PT, D), jnp.float32)
    ones_ebd = jnp.ones((EB, D), jnp.float32)

    cnts = _sc_counts(dst, zeros_nd, ones_ebd)
    e0 = _epre1(edge_attr, Wm0[D:], bm0.reshape(1, D))
    apre0 = _apre(x, Wm0[:D])

    part0 = _sc_edge(apre0, e0, src3d, dst3d, zeros_nd)
    e1 = _epre1(edge_attr, Wm1[D:], bm1.reshape(1, D))
    h1, apre1 = _update(part0, cnts, x, Wu0[:D], Wu0[D:],
                        bu0.reshape(1, D), Wm1[:D])
    part1 = _sc_edge(apre1, e1, src3d, dst3d, zeros_nd)
    e2 = _epre1(edge_attr, Wm2[D:], bm2.reshape(1, D))
    h2, apre2 = _update(part1, cnts, h1, Wu1[:D], Wu1[D:],
                        bu1.reshape(1, D), Wm2[:D])
    part2 = _sc_edge(apre2, e2, src3d, dst3d, zeros_nd)

    return _finale(part2, cnts, h2, Wu2[:D], Wu2[D:], bu2.reshape(1, D),
                   batch.reshape(1, N_NODES), W1, b1.reshape(1, D),
                   W2, b2.reshape(1, 64), W3, b3.reshape(1, 1))


# speedup vs baseline: 2.4238x; 1.2402x over previous
import jax
import jax.numpy as jnp
from jax.experimental import pallas as pl
from jax.experimental.pallas import tpu as pltpu
from jax.experimental.pallas import tpu_sc as plsc

N_NODES = 10000
N_EDGES = 320000
NUM_GRAPHS = 16
D = 128
DE = 16

NC = 2
NS = 16
LANES = 16
NW = NC * NS
EB = 128
NCHUNKS = N_EDGES // EB
ROWS_PT = 632
NROW_ACC = NS * ROWS_PT


def _leaky(v):
    return jnp.maximum(v, 0.2 * v)


def _epre1_body(ea_ref, w_ref, b_ref, o_ref):
    z = jax.lax.dot_general(ea_ref[...], w_ref[...],
                            (((0,), (0,)), ((), ())),
                            preferred_element_type=jnp.float32)
    o_ref[...] = z + b_ref[...]


def _epre1(edge_attr_t, w, b):
    BE = 3200
    return pl.pallas_call(
        _epre1_body,
        grid=(N_EDGES // BE,),
        in_specs=[
            pl.BlockSpec((DE, BE), lambda i: (0, i)),
            pl.BlockSpec((DE, D), lambda i: (0, 0)),
            pl.BlockSpec((1, D), lambda i: (0, 0)),
        ],
        out_specs=pl.BlockSpec((BE, D), lambda i: (i, 0)),
        out_shape=jax.ShapeDtypeStruct((N_EDGES, D), jnp.float32),
    )(edge_attr_t, w, b)


def _epre2_body(ea_ref, w_ref, b_ref, o1_ref, o2_ref):
    z = jnp.dot(ea_ref[...], w_ref[...], preferred_element_type=jnp.float32)
    z = z + b_ref[...]
    o1_ref[...] = z[:, :D]
    o2_ref[...] = z[:, D:]


def _epre2(edge_attr, w_cat, b_cat):
    BE = 2000
    out = jax.ShapeDtypeStruct((N_EDGES, D), jnp.float32)
    return pl.pallas_call(
        _epre2_body,
        grid=(N_EDGES // BE,),
        in_specs=[
            pl.BlockSpec((BE, DE), lambda i: (i, 0)),
            pl.BlockSpec((DE, 2 * D), lambda i: (0, 0)),
            pl.BlockSpec((1, 2 * D), lambda i: (0, 0)),
        ],
        out_specs=[
            pl.BlockSpec((BE, D), lambda i: (i, 0)),
            pl.BlockSpec((BE, D), lambda i: (i, 0)),
        ],
        out_shape=[out, out],
    )(edge_attr, w_cat, b_cat)


def _apre_body(h_ref, w_ref, o_ref):
    o_ref[...] = jnp.dot(h_ref[...], w_ref[...],
                         preferred_element_type=jnp.float32)


def _apre(h, w_top):
    return pl.pallas_call(
        _apre_body,
        out_shape=jax.ShapeDtypeStruct((N_NODES, D), jnp.float32),
    )(h, w_top)


def _update_body(p_ref, c_ref, h_ref, wt_ref, wb_ref, bu_ref, wn_ref,
                 oh_ref, oa_ref):
    cnt = c_ref[0, :, 0:1] + c_ref[1, :, 0:1]
    aggr = (p_ref[0] + p_ref[1]) / jnp.maximum(cnt, 1.0)
    z = jnp.dot(aggr, wt_ref[...], preferred_element_type=jnp.float32)
    z = z + jnp.dot(h_ref[...], wb_ref[...],
                    preferred_element_type=jnp.float32)
    hn = _leaky(z + bu_ref[...])
    oh_ref[...] = hn
    oa_ref[...] = jnp.dot(hn, wn_ref[...], preferred_element_type=jnp.float32)


def _update(partial, counts, h, wu_top, wu_bot, bu, w_next_top):
    return pl.pallas_call(
        _update_body,
        out_shape=[
            jax.ShapeDtypeStruct((N_NODES, D), jnp.float32),
            jax.ShapeDtypeStruct((N_NODES, D), jnp.float32),
        ],
    )(partial, counts, h, wu_top, wu_bot, bu, w_next_top)


def _finale_body(p_ref, c_ref, h_ref, wt_ref, wb_ref, bu_ref, b_ref,
                 w1_ref, b1_ref, w2_ref, b2_ref, w3_ref, b3_ref, o_ref):
    cnt = c_ref[0, :, 0:1] + c_ref[1, :, 0:1]
    aggr = (p_ref[0] + p_ref[1]) / jnp.maximum(cnt, 1.0)
    z = jnp.dot(aggr, wt_ref[...], preferred_element_type=jnp.float32)
    z = z + jnp.dot(h_ref[...], wb_ref[...],
                    preferred_element_type=jnp.float32)
    h3 = _leaky(z + bu_ref[...])
    gids = jax.lax.broadcasted_iota(jnp.int32, (NUM_GRAPHS, N_NODES), 0)
    mask = (b_ref[...] == gids).astype(jnp.float32)
    sums = jnp.dot(mask, h3, preferred_element_type=jnp.float32)
    gcnt = jnp.sum(mask, axis=1, keepdims=True)
    g = sums / jnp.maximum(gcnt, 1.0)
    g = _leaky(jnp.dot(g, w1_ref[...], preferred_element_type=jnp.float32)
               + b1_ref[...])
    g = _leaky(jnp.dot(g, w2_ref[...], preferred_element_type=jnp.float32)
               + b2_ref[...])
    o_ref[...] = (jnp.dot(g, w3_ref[...], preferred_element_type=jnp.float32)
                  + b3_ref[...])


def _finale(partial, counts, h, wu_top, wu_bot, bu, batch2d,
            w1, b1, w2, b2, w3, b3):
    return pl.pallas_call(
        _finale_body,
        out_shape=jax.ShapeDtypeStruct((NUM_GRAPHS, 1), jnp.float32),
    )(partial, counts, h, wu_top, wu_bot, bu, batch2d, w1, b1, w2, b2, w3, b3)


ECH = 64
IBK = 4
BLK = IBK * ECH
NBLK = N_EDGES // BLK
NITER = (NBLK + 2 * NW - 1) // (2 * NW)


def _sc_edge(apre, epre, src3d, dst3d, zeros_nd):
    mesh = plsc.VectorSubcoreMesh(core_axis_name="c", subcore_axis_name="s")

    def body(apre_hbm, epre_hbm, src_hbm, dst_hbm, z_hbm, out_hbm,
             gbuf0, ebuf0, seme0, semg0,
             gbuf1, ebuf1, seme1, semg1,
             sblk0, dblk0, sblk1, dblk1, acc):
        cid = jax.lax.axis_index("c")
        sid = jax.lax.axis_index("s")
        wid = sid * NC + cid
        row0 = pl.multiple_of(sid * ROWS_PT, 8)

        data = ((gbuf0, ebuf0, seme0, semg0), (gbuf1, ebuf1, seme1, semg1))
        iblk = ((sblk0, dblk0), (sblk1, dblk1))

        def load_iblk(b, ib):
            sblk, dblk = iblk[ib]
            pltpu.sync_copy(src_hbm.at[b], sblk)
            pltpu.sync_copy(dst_hbm.at[b], dblk)

        def start_chunk(b, c, ib, d):
            sblk, _ = iblk[ib]
            gbuf, ebuf, seme, semg = data[d]
            off = pl.multiple_of(b * BLK + c * ECH, 8)
            pltpu.make_async_copy(epre_hbm.at[pl.ds(off, ECH)], ebuf,
                                  seme).start()
            pltpu.make_async_copy(apre_hbm.at[sblk.at[c]], gbuf, semg).start()

        def finish_chunk(c, ib, d):
            sblk, dblk = iblk[ib]
            gbuf, ebuf, seme, semg = data[d]
            pltpu.make_async_copy(epre_hbm.at[pl.ds(0, ECH)], ebuf,
                                  seme).wait()
            pltpu.make_async_copy(apre_hbm.at[sblk.at[c]], gbuf, semg).wait()

            @pl.loop(0, ECH, step=4)
            def _row(i):
                for rr in range(4):
                    for j in range(D // LANES):
                        sl = pl.ds(j * LANES, LANES)
                        m = gbuf[i + rr, sl] + ebuf[i + rr, sl]
                        gbuf[i + rr, sl] = jnp.maximum(m, 0.2 * m)

            pltpu.sync_copy(gbuf, acc.at[dblk.at[c]], add=True)

        pltpu.sync_copy(z_hbm.at[sid], acc.at[pl.ds(row0, ROWS_PT)])
        plsc.subcore_barrier()

        load_iblk(wid, 0)
        start_chunk(wid, 0, 0, 0)

        @pl.loop(0, NITER)
        def _iter(t):
            b0 = wid + (2 * t) * NW
            b1 = b0 + NW
            bnext = b0 + 2 * NW
            gb1 = b1 < NBLK
            gnext = bnext < NBLK

            @pl.when(gb1)
            def _():
                load_iblk(b1, 1)

            start_chunk(b0, 1, 0, 1)
            finish_chunk(0, 0, 0)
            start_chunk(b0, 2, 0, 0)
            finish_chunk(1, 0, 1)
            start_chunk(b0, 3, 0, 1)
            finish_chunk(2, 0, 0)

            @pl.when(gb1)
            def _():
                start_chunk(b1, 0, 1, 0)

            finish_chunk(3, 0, 1)

            @pl.when(gb1)
            def _():
                start_chunk(b1, 1, 1, 1)
                finish_chunk(0, 1, 0)

                @pl.when(gnext)
                def _():
                    load_iblk(bnext, 0)

                start_chunk(b1, 2, 1, 0)
                finish_chunk(1, 1, 1)
                start_chunk(b1, 3, 1, 1)
                finish_chunk(2, 1, 0)

                @pl.when(gnext)
                def _():
                    start_chunk(bnext, 0, 0, 0)

                finish_chunk(3, 1, 1)

        plsc.subcore_barrier()
        pltpu.sync_copy(acc.at[pl.ds(row0, ROWS_PT)], out_hbm.at[cid, sid])

    k = pl.kernel(
        body,
        out_type=jax.ShapeDtypeStruct((NC, NS, ROWS_PT, D), jnp.float32),
        mesh=mesh,
        scratch_types=[
            pltpu.VMEM((ECH, D), jnp.float32),
            pltpu.VMEM((ECH, D), jnp.float32),
            pltpu.SemaphoreType.DMA,
            pltpu.SemaphoreType.DMA,
            pltpu.VMEM((ECH, D), jnp.float32),
            pltpu.VMEM((ECH, D), jnp.float32),
            pltpu.SemaphoreType.DMA,
            pltpu.SemaphoreType.DMA,
            pltpu.VMEM((IBK, ECH), jnp.int32),
            pltpu.VMEM((IBK, ECH), jnp.int32),
            pltpu.VMEM((IBK, ECH), jnp.int32),
            pltpu.VMEM((IBK, ECH), jnp.int32),
            pltpu.VMEM_SHARED((NROW_ACC, D), jnp.float32),
        ],
    )
    res = k(apre, epre, src3d, dst3d, zeros_nd)
    return res.reshape(NC, NROW_ACC, D)[:, :N_NODES]


def _sc_counts(dst, zeros_nd, ones_ebd):
    mesh = plsc.VectorSubcoreMesh(core_axis_name="c", subcore_axis_name="s")

    def body(dst_hbm, z_hbm, ones_hbm, cout_hbm, didx0, didx1, semi0, semi1,
             didxt, obuf, cacc):
        cid = jax.lax.axis_index("c")
        sid = jax.lax.axis_index("s")
        wid = sid * NC + cid
        row0 = pl.multiple_of(sid * ROWS_PT, 8)
        base = pl.multiple_of(wid * (N_EDGES // NW), 8)
        nch = (N_EDGES // NW) // EB

        ibufs = ((didx0, semi0), (didx1, semi1))

        def start_idx(ch, b):
            didx, semi = ibufs[b]
            off = pl.multiple_of(base + ch * EB, 8)
            pltpu.make_async_copy(dst_hbm.at[pl.ds(off, EB)], didx,
                                  semi).start()

        def finish(ch, b):
            didx, semi = ibufs[b]
            off = pl.multiple_of(base + ch * EB, 8)
            pltpu.make_async_copy(dst_hbm.at[pl.ds(off, EB)], didx,
                                  semi).wait()
            pltpu.sync_copy(obuf, cacc.at[didx], add=True)

        pltpu.sync_copy(z_hbm.at[sid], cacc.at[pl.ds(row0, ROWS_PT)])
        pltpu.sync_copy(ones_hbm, obuf)
        plsc.subcore_barrier()

        start_idx(0, 0)
        start_idx(1, 1)

        @pl.loop(0, nch // 2)
        def _pair(j):
            finish(2 * j, 0)

            @pl.when(j < nch // 2 - 1)
            def _():
                start_idx(2 * j + 2, 0)

            finish(2 * j + 1, 1)

            @pl.when(j < nch // 2 - 1)
            def _():
                start_idx(2 * j + 3, 1)

        offt = pl.multiple_of(base + nch * EB, 8)
        pltpu.sync_copy(dst_hbm.at[pl.ds(offt, 16)], didxt)
        pltpu.sync_copy(obuf.at[pl.ds(0, 16)], cacc.at[didxt], add=True)

        plsc.subcore_barrier()
        pltpu.sync_copy(cacc.at[pl.ds(row0, ROWS_PT)], cout_hbm.at[cid, sid])

    k = pl.kernel(
        body,
        out_type=jax.ShapeDtypeStruct((NC, NS, ROWS_PT, D), jnp.float32),
        mesh=mesh,
        scratch_types=[
            pltpu.VMEM((EB,), jnp.int32),
            pltpu.VMEM((EB,), jnp.int32),
            pltpu.SemaphoreType.DMA,
            pltpu.SemaphoreType.DMA,
            pltpu.VMEM((16,), jnp.int32),
            pltpu.VMEM((EB, D), jnp.float32),
            pltpu.VMEM_SHARED((NROW_ACC, D), jnp.float32),
        ],
    )
    res = k(dst, zeros_nd, ones_ebd)
    return res.reshape(NC, NROW_ACC, D)[:, :N_NODES, :LANES]


def kernel(x, edge_index, edge_attr, batch,
           Wm0, bm0, Wu0, bu0, Wm1, bm1, Wu1, bu1, Wm2, bm2, Wu2, bu2,
           W1, b1, W2, b2, W3, b3):
    src = edge_index[0]
    dst = edge_index[1]
    src3d = src.reshape(NBLK, IBK, ECH)
    dst3d = dst.reshape(NBLK, IBK, ECH)
    zeros_nd = jnp.zeros((NS, ROWS_PT, D), jnp.float32)
    ones_ebd = jnp.ones((EB, D), jnp.float32)

    cnts = _sc_counts(dst, zeros_nd, ones_ebd)
    ea_t = edge_attr.T
    e0 = _epre1(ea_t, Wm0[D:], bm0.reshape(1, D))
    apre0 = _apre(x, Wm0[:D])
    apre0, cnts = jax.lax.optimization_barrier((apre0, cnts))

    part0 = _sc_edge(apre0, e0, src3d, dst3d, zeros_nd)
    e1 = _epre1(ea_t, Wm1[D:], bm1.reshape(1, D))
    h1, apre1 = _update(part0, cnts, x, Wu0[:D], Wu0[D:],
                        bu0.reshape(1, D), Wm1[:D])
    part1 = _sc_edge(apre1, e1, src3d, dst3d, zeros_nd)
    e2 = _epre1(ea_t, Wm2[D:], bm2.reshape(1, D))
    h2, apre2 = _update(part1, cnts, h1, Wu1[:D], Wu1[D:],
                        bu1.reshape(1, D), Wm2[:D])
    part2 = _sc_edge(apre2, e2, src3d, dst3d, zeros_nd)

    return _finale(part2, cnts, h2, Wu2[:D], Wu2[D:], bu2.reshape(1, D),
                   batch.reshape(1, N_NODES), W1, b1.reshape(1, D),
                   W2, b2.reshape(1, 64), W3, b3.reshape(1, 1))


# speedup vs baseline: 2.4241x; 1.0001x over previous
import jax
import jax.numpy as jnp
from jax.experimental import pallas as pl
from jax.experimental.pallas import tpu as pltpu
from jax.experimental.pallas import tpu_sc as plsc

N_NODES = 10000
N_EDGES = 320000
NUM_GRAPHS = 16
D = 128
DE = 16

NC = 2
NS = 16
LANES = 16
NW = NC * NS
EB = 128
NCHUNKS = N_EDGES // EB
ROWS_PT = 632
NROW_ACC = NS * ROWS_PT


def _leaky(v):
    return jnp.maximum(v, 0.2 * v)


def _epre1_body(ea_ref, w_ref, b_ref, o_ref):
    z = jax.lax.dot_general(ea_ref[...], w_ref[...],
                            (((0,), (0,)), ((), ())),
                            precision=jax.lax.Precision.DEFAULT,
                            preferred_element_type=jnp.float32)
    o_ref[...] = z + b_ref[...]


def _epre1(edge_attr_t, w, b):
    BE = 3200
    return pl.pallas_call(
        _epre1_body,
        grid=(N_EDGES // BE,),
        in_specs=[
            pl.BlockSpec((DE, BE), lambda i: (0, i)),
            pl.BlockSpec((DE, D), lambda i: (0, 0)),
            pl.BlockSpec((1, D), lambda i: (0, 0)),
        ],
        out_specs=pl.BlockSpec((BE, D), lambda i: (i, 0)),
        out_shape=jax.ShapeDtypeStruct((N_EDGES, D), jnp.float32),
    )(edge_attr_t, w, b)


def _epre2_body(ea_ref, w_ref, b_ref, o1_ref, o2_ref):
    z = jnp.dot(ea_ref[...], w_ref[...], preferred_element_type=jnp.float32)
    z = z + b_ref[...]
    o1_ref[...] = z[:, :D]
    o2_ref[...] = z[:, D:]


def _epre2(edge_attr, w_cat, b_cat):
    BE = 2000
    out = jax.ShapeDtypeStruct((N_EDGES, D), jnp.float32)
    return pl.pallas_call(
        _epre2_body,
        grid=(N_EDGES // BE,),
        in_specs=[
            pl.BlockSpec((BE, DE), lambda i: (i, 0)),
            pl.BlockSpec((DE, 2 * D), lambda i: (0, 0)),
            pl.BlockSpec((1, 2 * D), lambda i: (0, 0)),
        ],
        out_specs=[
            pl.BlockSpec((BE, D), lambda i: (i, 0)),
            pl.BlockSpec((BE, D), lambda i: (i, 0)),
        ],
        out_shape=[out, out],
    )(edge_attr, w_cat, b_cat)


def _apre_body(h_ref, w_ref, o_ref):
    o_ref[...] = jnp.dot(h_ref[...], w_ref[...],
                         preferred_element_type=jnp.float32)


def _apre(h, w_top):
    return pl.pallas_call(
        _apre_body,
        out_shape=jax.ShapeDtypeStruct((N_NODES, D), jnp.float32),
    )(h, w_top)


def _update_body(p_ref, c_ref, h_ref, wt_ref, wb_ref, bu_ref, wn_ref,
                 oh_ref, oa_ref):
    cnt = c_ref[0, :, 0:1] + c_ref[1, :, 0:1]
    aggr = (p_ref[0] + p_ref[1]) / jnp.maximum(cnt, 1.0)
    z = jnp.dot(aggr, wt_ref[...], preferred_element_type=jnp.float32)
    z = z + jnp.dot(h_ref[...], wb_ref[...],
                    preferred_element_type=jnp.float32)
    hn = _leaky(z + bu_ref[...])
    oh_ref[...] = hn
    oa_ref[...] = jnp.dot(hn, wn_ref[...], preferred_element_type=jnp.float32)


def _update(partial, counts, h, wu_top, wu_bot, bu, w_next_top):
    return pl.pallas_call(
        _update_body,
        out_shape=[
            jax.ShapeDtypeStruct((N_NODES, D), jnp.float32),
            jax.ShapeDtypeStruct((N_NODES, D), jnp.float32),
        ],
    )(partial, counts, h, wu_top, wu_bot, bu, w_next_top)


def _finale_body(p_ref, c_ref, h_ref, wt_ref, wb_ref, bu_ref, b_ref,
                 w1_ref, b1_ref, w2_ref, b2_ref, w3_ref, b3_ref, o_ref):
    cnt = c_ref[0, :, 0:1] + c_ref[1, :, 0:1]
    aggr = (p_ref[0] + p_ref[1]) / jnp.maximum(cnt, 1.0)
    z = jnp.dot(aggr, wt_ref[...], preferred_element_type=jnp.float32)
    z = z + jnp.dot(h_ref[...], wb_ref[...],
                    preferred_element_type=jnp.float32)
    h3 = _leaky(z + bu_ref[...])
    gids = jax.lax.broadcasted_iota(jnp.int32, (NUM_GRAPHS, N_NODES), 0)
    mask = (b_ref[...] == gids).astype(jnp.float32)
    sums = jnp.dot(mask, h3, preferred_element_type=jnp.float32)
    gcnt = jnp.sum(mask, axis=1, keepdims=True)
    g = sums / jnp.maximum(gcnt, 1.0)
    g = _leaky(jnp.dot(g, w1_ref[...], preferred_element_type=jnp.float32)
               + b1_ref[...])
    g = _leaky(jnp.dot(g, w2_ref[...], preferred_element_type=jnp.float32)
               + b2_ref[...])
    o_ref[...] = (jnp.dot(g, w3_ref[...], preferred_element_type=jnp.float32)
                  + b3_ref[...])


def _finale(partial, counts, h, wu_top, wu_bot, bu, batch2d,
            w1, b1, w2, b2, w3, b3):
    return pl.pallas_call(
        _finale_body,
        out_shape=jax.ShapeDtypeStruct((NUM_GRAPHS, 1), jnp.float32),
    )(partial, counts, h, wu_top, wu_bot, bu, batch2d, w1, b1, w2, b2, w3, b3)


ECH = 64
IBK = 4
BLK = IBK * ECH
NBLK = N_EDGES // BLK
NITER = (NBLK + 2 * NW - 1) // (2 * NW)


def _sc_edge(apre, epre, src3d, dst3d, zeros_nd):
    mesh = plsc.VectorSubcoreMesh(core_axis_name="c", subcore_axis_name="s")

    def body(apre_hbm, epre_hbm, src_hbm, dst_hbm, z_hbm, out_hbm,
             gbuf0, ebuf0, seme0, semg0,
             gbuf1, ebuf1, seme1, semg1,
             sblk0, dblk0, sblk1, dblk1, acc):
        cid = jax.lax.axis_index("c")
        sid = jax.lax.axis_index("s")
        wid = sid * NC + cid
        row0 = pl.multiple_of(sid * ROWS_PT, 8)

        data = ((gbuf0, ebuf0, seme0, semg0), (gbuf1, ebuf1, seme1, semg1))
        iblk = ((sblk0, dblk0), (sblk1, dblk1))

        def load_iblk(b, ib):
            sblk, dblk = iblk[ib]
            pltpu.sync_copy(src_hbm.at[b], sblk)
            pltpu.sync_copy(dst_hbm.at[b], dblk)

        def start_chunk(b, c, ib, d):
            sblk, _ = iblk[ib]
            gbuf, ebuf, seme, semg = data[d]
            off = pl.multiple_of(b * BLK + c * ECH, 8)
            pltpu.make_async_copy(epre_hbm.at[pl.ds(off, ECH)], ebuf,
                                  seme).start()
            pltpu.make_async_copy(apre_hbm.at[sblk.at[c]], gbuf, semg).start()

        def finish_chunk(c, ib, d):
            sblk, dblk = iblk[ib]
            gbuf, ebuf, seme, semg = data[d]
            pltpu.make_async_copy(epre_hbm.at[pl.ds(0, ECH)], ebuf,
                                  seme).wait()
            pltpu.make_async_copy(apre_hbm.at[sblk.at[c]], gbuf, semg).wait()

            @pl.loop(0, ECH, step=4)
            def _row(i):
                for rr in range(4):
                    for j in range(D // LANES):
                        sl = pl.ds(j * LANES, LANES)
                        m = gbuf[i + rr, sl] + ebuf[i + rr, sl]
                        gbuf[i + rr, sl] = jnp.maximum(m, 0.2 * m)

            pltpu.sync_copy(gbuf, acc.at[dblk.at[c]], add=True)

        pltpu.sync_copy(z_hbm.at[sid], acc.at[pl.ds(row0, ROWS_PT)])
        plsc.subcore_barrier()

        load_iblk(wid, 0)
        start_chunk(wid, 0, 0, 0)

        @pl.loop(0, NITER)
        def _iter(t):
            b0 = wid + (2 * t) * NW
            b1 = b0 + NW
            bnext = b0 + 2 * NW
            gb1 = b1 < NBLK
            gnext = bnext < NBLK

            @pl.when(gb1)
            def _():
                load_iblk(b1, 1)

            start_chunk(b0, 1, 0, 1)
            finish_chunk(0, 0, 0)
            start_chunk(b0, 2, 0, 0)
            finish_chunk(1, 0, 1)
            start_chunk(b0, 3, 0, 1)
            finish_chunk(2, 0, 0)

            @pl.when(gb1)
            def _():
                start_chunk(b1, 0, 1, 0)

            finish_chunk(3, 0, 1)

            @pl.when(gb1)
            def _():
                start_chunk(b1, 1, 1, 1)
                finish_chunk(0, 1, 0)

                @pl.when(gnext)
                def _():
                    load_iblk(bnext, 0)

                start_chunk(b1, 2, 1, 0)
                finish_chunk(1, 1, 1)
                start_chunk(b1, 3, 1, 1)
                finish_chunk(2, 1, 0)

                @pl.when(gnext)
                def _():
                    start_chunk(bnext, 0, 0, 0)

                finish_chunk(3, 1, 1)

        plsc.subcore_barrier()
        pltpu.sync_copy(acc.at[pl.ds(row0, ROWS_PT)], out_hbm.at[cid, sid])

    k = pl.kernel(
        body,
        out_type=jax.ShapeDtypeStruct((NC, NS, ROWS_PT, D), jnp.float32),
        mesh=mesh,
        scratch_types=[
            pltpu.VMEM((ECH, D), jnp.float32),
            pltpu.VMEM((ECH, D), jnp.float32),
            pltpu.SemaphoreType.DMA,
            pltpu.SemaphoreType.DMA,
            pltpu.VMEM((ECH, D), jnp.float32),
            pltpu.VMEM((ECH, D), jnp.float32),
            pltpu.SemaphoreType.DMA,
            pltpu.SemaphoreType.DMA,
            pltpu.VMEM((IBK, ECH), jnp.int32),
            pltpu.VMEM((IBK, ECH), jnp.int32),
            pltpu.VMEM((IBK, ECH), jnp.int32),
            pltpu.VMEM((IBK, ECH), jnp.int32),
            pltpu.VMEM_SHARED((NROW_ACC, D), jnp.float32),
        ],
    )
    res = k(apre, epre, src3d, dst3d, zeros_nd)
    return res.reshape(NC, NROW_ACC, D)[:, :N_NODES]


def _sc_counts(dst, zeros_nd, ones_ebd):
    mesh = plsc.VectorSubcoreMesh(core_axis_name="c", subcore_axis_name="s")

    def body(dst_hbm, z_hbm, ones_hbm, cout_hbm, didx0, didx1, semi0, semi1,
             didxt, obuf, cacc):
        cid = jax.lax.axis_index("c")
        sid = jax.lax.axis_index("s")
        wid = sid * NC + cid
        row0 = pl.multiple_of(sid * ROWS_PT, 8)
        base = pl.multiple_of(wid * (N_EDGES // NW), 8)
        nch = (N_EDGES // NW) // EB

        ibufs = ((didx0, semi0), (didx1, semi1))

        def start_idx(ch, b):
            didx, semi = ibufs[b]
            off = pl.multiple_of(base + ch * EB, 8)
            pltpu.make_async_copy(dst_hbm.at[pl.ds(off, EB)], didx,
                                  semi).start()

        def finish(ch, b):
            didx, semi = ibufs[b]
            off = pl.multiple_of(base + ch * EB, 8)
            pltpu.make_async_copy(dst_hbm.at[pl.ds(off, EB)], didx,
                                  semi).wait()
            pltpu.sync_copy(obuf, cacc.at[didx], add=True)

        pltpu.sync_copy(z_hbm.at[sid], cacc.at[pl.ds(row0, ROWS_PT)])
        pltpu.sync_copy(ones_hbm, obuf)
        plsc.subcore_barrier()

        start_idx(0, 0)
        start_idx(1, 1)

        @pl.loop(0, nch // 2)
        def _pair(j):
            finish(2 * j, 0)

            @pl.when(j < nch // 2 - 1)
            def _():
                start_idx(2 * j + 2, 0)

            finish(2 * j + 1, 1)

            @pl.when(j < nch // 2 - 1)
            def _():
                start_idx(2 * j + 3, 1)

        offt = pl.multiple_of(base + nch * EB, 8)
        pltpu.sync_copy(dst_hbm.at[pl.ds(offt, 16)], didxt)
        pltpu.sync_copy(obuf.at[pl.ds(0, 16)], cacc.at[didxt], add=True)

        plsc.subcore_barrier()
        pltpu.sync_copy(cacc.at[pl.ds(row0, ROWS_PT)], cout_hbm.at[cid, sid])

    k = pl.kernel(
        body,
        out_type=jax.ShapeDtypeStruct((NC, NS, ROWS_PT, D), jnp.float32),
        mesh=mesh,
        scratch_types=[
            pltpu.VMEM((EB,), jnp.int32),
            pltpu.VMEM((EB,), jnp.int32),
            pltpu.SemaphoreType.DMA,
            pltpu.SemaphoreType.DMA,
            pltpu.VMEM((16,), jnp.int32),
            pltpu.VMEM((EB, D), jnp.float32),
            pltpu.VMEM_SHARED((NROW_ACC, D), jnp.float32),
        ],
    )
    res = k(dst, zeros_nd, ones_ebd)
    return res.reshape(NC, NROW_ACC, D)[:, :N_NODES, :LANES]


def kernel(x, edge_index, edge_attr, batch,
           Wm0, bm0, Wu0, bu0, Wm1, bm1, Wu1, bu1, Wm2, bm2, Wu2, bu2,
           W1, b1, W2, b2, W3, b3):
    src = edge_index[0]
    dst = edge_index[1]
    src3d = src.reshape(NBLK, IBK, ECH)
    dst3d = dst.reshape(NBLK, IBK, ECH)
    zeros_nd = jnp.zeros((NS, ROWS_PT, D), jnp.float32)
    ones_ebd = jnp.ones((EB, D), jnp.float32)

    cnts = _sc_counts(dst, zeros_nd, ones_ebd)
    ea_t = edge_attr.T
    e0 = _epre1(ea_t, Wm0[D:], bm0.reshape(1, D))
    apre0 = _apre(x, Wm0[:D])
    apre0, cnts = jax.lax.optimization_barrier((apre0, cnts))

    part0 = _sc_edge(apre0, e0, src3d, dst3d, zeros_nd)
    e1 = _epre1(ea_t, Wm1[D:], bm1.reshape(1, D))
    h1, apre1 = _update(part0, cnts, x, Wu0[:D], Wu0[D:],
                        bu0.reshape(1, D), Wm1[:D])
    part1 = _sc_edge(apre1, e1, src3d, dst3d, zeros_nd)
    e2 = _epre1(ea_t, Wm2[D:], bm2.reshape(1, D))
    h2, apre2 = _update(part1, cnts, h1, Wu1[:D], Wu1[D:],
                        bu1.reshape(1, D), Wm2[:D])
    part2 = _sc_edge(apre2, e2, src3d, dst3d, zeros_nd)

    return _finale(part2, cnts, h2, Wu2[:D], Wu2[D:], bu2.reshape(1, D),
                   batch.reshape(1, N_NODES), W1, b1.reshape(1, D),
                   W2, b2.reshape(1, 64), W3, b3.reshape(1, 1))


# speedup vs baseline: 2.4789x; 1.0226x over previous
import jax
import jax.numpy as jnp
from jax.experimental import pallas as pl
from jax.experimental.pallas import tpu as pltpu
from jax.experimental.pallas import tpu_sc as plsc

N_NODES = 10000
N_EDGES = 320000
NUM_GRAPHS = 16
D = 128
DE = 16

NC = 2
NS = 16
LANES = 16
NW = NC * NS
EB = 128
NCHUNKS = N_EDGES // EB
ROWS_PT = 632
NROW_ACC = NS * ROWS_PT


def _leaky(v):
    return jnp.maximum(v, 0.2 * v)


def _epre1_body(ea_ref, w_ref, b_ref, o_ref):
    z = jax.lax.dot_general(ea_ref[...], w_ref[...],
                            (((0,), (0,)), ((), ())),
                            preferred_element_type=jnp.float32)
    o_ref[...] = z + b_ref[...]


def _epre1(edge_attr_t, w, b):
    BE = 3200
    return pl.pallas_call(
        _epre1_body,
        grid=(N_EDGES // BE,),
        in_specs=[
            pl.BlockSpec((DE, BE), lambda i: (0, i)),
            pl.BlockSpec((DE, D), lambda i: (0, 0)),
            pl.BlockSpec((1, D), lambda i: (0, 0)),
        ],
        out_specs=pl.BlockSpec((BE, D), lambda i: (i, 0)),
        out_shape=jax.ShapeDtypeStruct((N_EDGES, D), jnp.float32),
    )(edge_attr_t, w, b)


def _epre2_body(ea_ref, w_ref, b_ref, o1_ref, o2_ref):
    z = jnp.dot(ea_ref[...], w_ref[...], preferred_element_type=jnp.float32)
    z = z + b_ref[...]
    o1_ref[...] = z[:, :D]
    o2_ref[...] = z[:, D:]


def _epre2(edge_attr, w_cat, b_cat):
    BE = 2000
    out = jax.ShapeDtypeStruct((N_EDGES, D), jnp.float32)
    return pl.pallas_call(
        _epre2_body,
        grid=(N_EDGES // BE,),
        in_specs=[
            pl.BlockSpec((BE, DE), lambda i: (i, 0)),
            pl.BlockSpec((DE, 2 * D), lambda i: (0, 0)),
            pl.BlockSpec((1, 2 * D), lambda i: (0, 0)),
        ],
        out_specs=[
            pl.BlockSpec((BE, D), lambda i: (i, 0)),
            pl.BlockSpec((BE, D), lambda i: (i, 0)),
        ],
        out_shape=[out, out],
    )(edge_attr, w_cat, b_cat)


def _apre_body(h_ref, w_ref, o_ref):
    o_ref[...] = jnp.dot(h_ref[...], w_ref[...],
                         preferred_element_type=jnp.float32)


def _apre(h, w_top):
    return pl.pallas_call(
        _apre_body,
        out_shape=jax.ShapeDtypeStruct((N_NODES, D), jnp.float32),
    )(h, w_top)


def _update_body(p_ref, c_ref, h_ref, wt_ref, wb_ref, bu_ref, wn_ref,
                 oh_ref, oa_ref):
    cnt = c_ref[0, :, 0:1] + c_ref[1, :, 0:1]
    aggr = (p_ref[0] + p_ref[1]) / jnp.maximum(cnt, 1.0)
    z = jnp.dot(aggr, wt_ref[...], preferred_element_type=jnp.float32)
    z = z + jnp.dot(h_ref[...], wb_ref[...],
                    preferred_element_type=jnp.float32)
    hn = _leaky(z + bu_ref[...])
    oh_ref[...] = hn
    oa_ref[...] = jnp.dot(hn, wn_ref[...], preferred_element_type=jnp.float32)


_PAD_SPECS = [
    pl.BlockSpec((NC, N_NODES, D), lambda i: (0, 0, 0)),
    pl.BlockSpec((NC, N_NODES, D), lambda i: (0, 0, 0)),
]


def _full_spec(x):
    return pl.BlockSpec(x.shape, lambda i: (0,) * x.ndim)


def _update(partial, counts, h, wu_top, wu_bot, bu, w_next_top):
    return pl.pallas_call(
        _update_body,
        grid=(1,),
        in_specs=_PAD_SPECS + [_full_spec(x)
                               for x in (h, wu_top, wu_bot, bu, w_next_top)],
        out_specs=[
            pl.BlockSpec((N_NODES, D), lambda i: (0, 0)),
            pl.BlockSpec((N_NODES, D), lambda i: (0, 0)),
        ],
        out_shape=[
            jax.ShapeDtypeStruct((N_NODES, D), jnp.float32),
            jax.ShapeDtypeStruct((N_NODES, D), jnp.float32),
        ],
    )(partial, counts, h, wu_top, wu_bot, bu, w_next_top)


def _finale_body(p_ref, c_ref, h_ref, wt_ref, wb_ref, bu_ref, b_ref,
                 w1_ref, b1_ref, w2_ref, b2_ref, w3_ref, b3_ref, o_ref):
    cnt = c_ref[0, :, 0:1] + c_ref[1, :, 0:1]
    aggr = (p_ref[0] + p_ref[1]) / jnp.maximum(cnt, 1.0)
    z = jnp.dot(aggr, wt_ref[...], preferred_element_type=jnp.float32)
    z = z + jnp.dot(h_ref[...], wb_ref[...],
                    preferred_element_type=jnp.float32)
    h3 = _leaky(z + bu_ref[...])
    gids = jax.lax.broadcasted_iota(jnp.int32, (NUM_GRAPHS, N_NODES), 0)
    mask = (b_ref[...] == gids).astype(jnp.float32)
    sums = jnp.dot(mask, h3, preferred_element_type=jnp.float32)
    gcnt = jnp.sum(mask, axis=1, keepdims=True)
    g = sums / jnp.maximum(gcnt, 1.0)
    g = _leaky(jnp.dot(g, w1_ref[...], preferred_element_type=jnp.float32)
               + b1_ref[...])
    g = _leaky(jnp.dot(g, w2_ref[...], preferred_element_type=jnp.float32)
               + b2_ref[...])
    o_ref[...] = (jnp.dot(g, w3_ref[...], preferred_element_type=jnp.float32)
                  + b3_ref[...])


def _finale(partial, counts, h, wu_top, wu_bot, bu, batch2d,
            w1, b1, w2, b2, w3, b3):
    rest = (h, wu_top, wu_bot, bu, batch2d, w1, b1, w2, b2, w3, b3)
    return pl.pallas_call(
        _finale_body,
        grid=(1,),
        in_specs=_PAD_SPECS + [_full_spec(x) for x in rest],
        out_specs=pl.BlockSpec((NUM_GRAPHS, 1), lambda i: (0, 0)),
        out_shape=jax.ShapeDtypeStruct((NUM_GRAPHS, 1), jnp.float32),
    )(partial, counts, *rest)


ECH = 64
IBK = 4
BLK = IBK * ECH
NBLK = N_EDGES // BLK
NITER = (NBLK + 2 * NW - 1) // (2 * NW)


def _sc_edge(apre, epre, src3d, dst3d, zeros_nd):
    mesh = plsc.VectorSubcoreMesh(core_axis_name="c", subcore_axis_name="s")

    def body(apre_hbm, epre_hbm, src_hbm, dst_hbm, z_hbm, out_hbm,
             gbuf0, ebuf0, seme0, semg0,
             gbuf1, ebuf1, seme1, semg1,
             sblk0, dblk0, sblk1, dblk1, acc):
        cid = jax.lax.axis_index("c")
        sid = jax.lax.axis_index("s")
        wid = sid * NC + cid
        row0 = pl.multiple_of(sid * ROWS_PT, 8)

        data = ((gbuf0, ebuf0, seme0, semg0), (gbuf1, ebuf1, seme1, semg1))
        iblk = ((sblk0, dblk0), (sblk1, dblk1))

        def load_iblk(b, ib):
            sblk, dblk = iblk[ib]
            pltpu.sync_copy(src_hbm.at[b], sblk)
            pltpu.sync_copy(dst_hbm.at[b], dblk)

        def start_chunk(b, c, ib, d):
            sblk, _ = iblk[ib]
            gbuf, ebuf, seme, semg = data[d]
            off = pl.multiple_of(b * BLK + c * ECH, 8)
            pltpu.make_async_copy(epre_hbm.at[pl.ds(off, ECH)], ebuf,
                                  seme).start()
            pltpu.make_async_copy(apre_hbm.at[sblk.at[c]], gbuf, semg).start()

        def finish_chunk(c, ib, d):
            sblk, dblk = iblk[ib]
            gbuf, ebuf, seme, semg = data[d]
            pltpu.make_async_copy(epre_hbm.at[pl.ds(0, ECH)], ebuf,
                                  seme).wait()
            pltpu.make_async_copy(apre_hbm.at[sblk.at[c]], gbuf, semg).wait()

            @pl.loop(0, ECH, step=4)
            def _row(i):
                for rr in range(4):
                    for j in range(D // LANES):
                        sl = pl.ds(j * LANES, LANES)
                        m = gbuf[i + rr, sl] + ebuf[i + rr, sl]
                        gbuf[i + rr, sl] = jnp.maximum(m, 0.2 * m)

            pltpu.sync_copy(gbuf, acc.at[dblk.at[c]], add=True)

        pltpu.sync_copy(z_hbm.at[sid], acc.at[pl.ds(row0, ROWS_PT)])
        plsc.subcore_barrier()

        load_iblk(wid, 0)
        start_chunk(wid, 0, 0, 0)

        @pl.loop(0, NITER)
        def _iter(t):
            b0 = wid + (2 * t) * NW
            b1 = b0 + NW
            bnext = b0 + 2 * NW
            gb1 = b1 < NBLK
            gnext = bnext < NBLK

            @pl.when(gb1)
            def _():
                load_iblk(b1, 1)

            start_chunk(b0, 1, 0, 1)
            finish_chunk(0, 0, 0)
            start_chunk(b0, 2, 0, 0)
            finish_chunk(1, 0, 1)
            start_chunk(b0, 3, 0, 1)
            finish_chunk(2, 0, 0)

            @pl.when(gb1)
            def _():
                start_chunk(b1, 0, 1, 0)

            finish_chunk(3, 0, 1)

            @pl.when(gb1)
            def _():
                start_chunk(b1, 1, 1, 1)
                finish_chunk(0, 1, 0)

                @pl.when(gnext)
                def _():
                    load_iblk(bnext, 0)

                start_chunk(b1, 2, 1, 0)
                finish_chunk(1, 1, 1)
                start_chunk(b1, 3, 1, 1)
                finish_chunk(2, 1, 0)

                @pl.when(gnext)
                def _():
                    start_chunk(bnext, 0, 0, 0)

                finish_chunk(3, 1, 1)

        plsc.subcore_barrier()
        pltpu.sync_copy(acc.at[pl.ds(row0, ROWS_PT)], out_hbm.at[cid, sid])

    k = pl.kernel(
        body,
        out_type=jax.ShapeDtypeStruct((NC, NS, ROWS_PT, D), jnp.float32),
        mesh=mesh,
        scratch_types=[
            pltpu.VMEM((ECH, D), jnp.float32),
            pltpu.VMEM((ECH, D), jnp.float32),
            pltpu.SemaphoreType.DMA,
            pltpu.SemaphoreType.DMA,
            pltpu.VMEM((ECH, D), jnp.float32),
            pltpu.VMEM((ECH, D), jnp.float32),
            pltpu.SemaphoreType.DMA,
            pltpu.SemaphoreType.DMA,
            pltpu.VMEM((IBK, ECH), jnp.int32),
            pltpu.VMEM((IBK, ECH), jnp.int32),
            pltpu.VMEM((IBK, ECH), jnp.int32),
            pltpu.VMEM((IBK, ECH), jnp.int32),
            pltpu.VMEM_SHARED((NROW_ACC, D), jnp.float32),
        ],
    )
    res = k(apre, epre, src3d, dst3d, zeros_nd)
    return res.reshape(NC, NROW_ACC, D)


def _sc_counts(dst, zeros_nd, ones_ebd):
    mesh = plsc.VectorSubcoreMesh(core_axis_name="c", subcore_axis_name="s")

    def body(dst_hbm, z_hbm, ones_hbm, cout_hbm, didx0, didx1, semi0, semi1,
             didxt, obuf, cacc):
        cid = jax.lax.axis_index("c")
        sid = jax.lax.axis_index("s")
        wid = sid * NC + cid
        row0 = pl.multiple_of(sid * ROWS_PT, 8)
        base = pl.multiple_of(wid * (N_EDGES // NW), 8)
        nch = (N_EDGES // NW) // EB

        ibufs = ((didx0, semi0), (didx1, semi1))

        def start_idx(ch, b):
            didx, semi = ibufs[b]
            off = pl.multiple_of(base + ch * EB, 8)
            pltpu.make_async_copy(dst_hbm.at[pl.ds(off, EB)], didx,
                                  semi).start()

        def finish(ch, b):
            didx, semi = ibufs[b]
            off = pl.multiple_of(base + ch * EB, 8)
            pltpu.make_async_copy(dst_hbm.at[pl.ds(off, EB)], didx,
                                  semi).wait()
            pltpu.sync_copy(obuf, cacc.at[didx], add=True)

        pltpu.sync_copy(z_hbm.at[sid], cacc.at[pl.ds(row0, ROWS_PT)])
        pltpu.sync_copy(ones_hbm, obuf)
        plsc.subcore_barrier()

        start_idx(0, 0)
        start_idx(1, 1)

        @pl.loop(0, nch // 2)
        def _pair(j):
            finish(2 * j, 0)

            @pl.when(j < nch // 2 - 1)
            def _():
                start_idx(2 * j + 2, 0)

            finish(2 * j + 1, 1)

            @pl.when(j < nch // 2 - 1)
            def _():
                start_idx(2 * j + 3, 1)

        offt = pl.multiple_of(base + nch * EB, 8)
        pltpu.sync_copy(dst_hbm.at[pl.ds(offt, 16)], didxt)
        pltpu.sync_copy(obuf.at[pl.ds(0, 16)], cacc.at[didxt], add=True)

        plsc.subcore_barrier()
        pltpu.sync_copy(cacc.at[pl.ds(row0, ROWS_PT)], cout_hbm.at[cid, sid])

    k = pl.kernel(
        body,
        out_type=jax.ShapeDtypeStruct((NC, NS, ROWS_PT, D), jnp.float32),
        mesh=mesh,
        scratch_types=[
            pltpu.VMEM((EB,), jnp.int32),
            pltpu.VMEM((EB,), jnp.int32),
            pltpu.SemaphoreType.DMA,
            pltpu.SemaphoreType.DMA,
            pltpu.VMEM((16,), jnp.int32),
            pltpu.VMEM((EB, D), jnp.float32),
            pltpu.VMEM_SHARED((NROW_ACC, D), jnp.float32),
        ],
    )
    res = k(dst, zeros_nd, ones_ebd)
    return res.reshape(NC, NROW_ACC, D)[:, :N_NODES, :LANES]


def kernel(x, edge_index, edge_attr, batch,
           Wm0, bm0, Wu0, bu0, Wm1, bm1, Wu1, bu1, Wm2, bm2, Wu2, bu2,
           W1, b1, W2, b2, W3, b3):
    src = edge_index[0]
    dst = edge_index[1]
    src3d = src.reshape(NBLK, IBK, ECH)
    dst3d = dst.reshape(NBLK, IBK, ECH)
    zeros_nd = jnp.zeros((NS, ROWS_PT, D), jnp.float32)
    ones_ebd = jnp.ones((EB, D), jnp.float32)

    cnts = _sc_counts(dst, zeros_nd, ones_ebd)
    ea_t = edge_attr.T
    e0 = _epre1(ea_t, Wm0[D:], bm0.reshape(1, D))
    apre0 = _apre(x, Wm0[:D])
    apre0, cnts = jax.lax.optimization_barrier((apre0, cnts))

    part0 = _sc_edge(apre0, e0, src3d, dst3d, zeros_nd)
    e1 = _epre1(ea_t, Wm1[D:], bm1.reshape(1, D))
    h1, apre1 = _update(part0, cnts, x, Wu0[:D], Wu0[D:],
                        bu0.reshape(1, D), Wm1[:D])
    part1 = _sc_edge(apre1, e1, src3d, dst3d, zeros_nd)
    e2 = _epre1(ea_t, Wm2[D:], bm2.reshape(1, D))
    h2, apre2 = _update(part1, cnts, h1, Wu1[:D], Wu1[D:],
                        bu1.reshape(1, D), Wm2[:D])
    part2 = _sc_edge(apre2, e2, src3d, dst3d, zeros_nd)

    return _finale(part2, cnts, h2, Wu2[:D], Wu2[D:], bu2.reshape(1, D),
                   batch.reshape(1, N_NODES), W1, b1.reshape(1, D),
                   W2, b2.reshape(1, 64), W3, b3.reshape(1, 1))


# speedup vs baseline: 2.4817x; 1.0011x over previous
import jax
import jax.numpy as jnp
from jax.experimental import pallas as pl
from jax.experimental.pallas import tpu as pltpu
from jax.experimental.pallas import tpu_sc as plsc

N_NODES = 10000
N_EDGES = 320000
NUM_GRAPHS = 16
D = 128
DE = 16

NC = 2
NS = 16
LANES = 16
NW = NC * NS
EB = 128
NCHUNKS = N_EDGES // EB
ROWS_PT = 632
NROW_ACC = NS * ROWS_PT


def _leaky(v):
    return jnp.maximum(v, 0.2 * v)


def _epre1_body(ea_ref, w_ref, b_ref, o_ref):
    z = jax.lax.dot_general(ea_ref[...], w_ref[...],
                            (((0,), (0,)), ((), ())),
                            preferred_element_type=jnp.float32)
    o_ref[...] = z + b_ref[...]


def _epre1(edge_attr_t, w, b):
    BE = 3200
    return pl.pallas_call(
        _epre1_body,
        grid=(N_EDGES // BE,),
        in_specs=[
            pl.BlockSpec((DE, BE), lambda i: (0, i)),
            pl.BlockSpec((DE, D), lambda i: (0, 0)),
            pl.BlockSpec((1, D), lambda i: (0, 0)),
        ],
        out_specs=pl.BlockSpec((BE, D), lambda i: (i, 0)),
        out_shape=jax.ShapeDtypeStruct((N_EDGES, D), jnp.float32),
    )(edge_attr_t, w, b)


def _apre_body(h_ref, w_ref, o_ref):
    o_ref[...] = jnp.dot(h_ref[...], w_ref[...],
                         preferred_element_type=jnp.float32)


def _apre(h, w_top):
    return pl.pallas_call(
        _apre_body,
        out_shape=jax.ShapeDtypeStruct((N_NODES, D), jnp.float32),
    )(h, w_top)


def _update_body(p_ref, c_ref, h_ref, wt_ref, wb_ref, bu_ref, wn_ref,
                 oh_ref, oa_ref):
    cnt = c_ref[0, :, 0:1] + c_ref[1, :, 0:1]
    aggr = (p_ref[0] + p_ref[1]) / jnp.maximum(cnt, 1.0)
    z = jnp.dot(aggr, wt_ref[...], preferred_element_type=jnp.float32)
    z = z + jnp.dot(h_ref[...], wb_ref[...],
                    preferred_element_type=jnp.float32)
    hn = _leaky(z + bu_ref[...])
    oh_ref[...] = hn
    oa_ref[...] = jnp.dot(hn, wn_ref[...], preferred_element_type=jnp.float32)


_PAD_SPECS = [
    pl.BlockSpec((NC, N_NODES, D), lambda i: (0, 0, 0)),
    pl.BlockSpec((NC, N_NODES, D), lambda i: (0, 0, 0)),
]


def _full_spec(x):
    return pl.BlockSpec(x.shape, lambda i: (0,) * x.ndim)


def _update(partial, counts, h, wu_top, wu_bot, bu, w_next_top):
    return pl.pallas_call(
        _update_body,
        grid=(1,),
        in_specs=_PAD_SPECS + [_full_spec(x)
                               for x in (h, wu_top, wu_bot, bu, w_next_top)],
        out_specs=[
            pl.BlockSpec((N_NODES, D), lambda i: (0, 0)),
            pl.BlockSpec((N_NODES, D), lambda i: (0, 0)),
        ],
        out_shape=[
            jax.ShapeDtypeStruct((N_NODES, D), jnp.float32),
            jax.ShapeDtypeStruct((N_NODES, D), jnp.float32),
        ],
    )(partial, counts, h, wu_top, wu_bot, bu, w_next_top)


def _finale_body(p_ref, c_ref, h_ref, wt_ref, wb_ref, bu_ref, b_ref,
                 w1_ref, b1_ref, w2_ref, b2_ref, w3_ref, b3_ref, o_ref):
    cnt = c_ref[0, :, 0:1] + c_ref[1, :, 0:1]
    aggr = (p_ref[0] + p_ref[1]) / jnp.maximum(cnt, 1.0)
    z = jnp.dot(aggr, wt_ref[...], preferred_element_type=jnp.float32)
    z = z + jnp.dot(h_ref[...], wb_ref[...],
                    preferred_element_type=jnp.float32)
    h3 = _leaky(z + bu_ref[...])
    gids = jax.lax.broadcasted_iota(jnp.int32, (NUM_GRAPHS, N_NODES), 0)
    mask = (b_ref[...] == gids).astype(jnp.float32)
    sums = jnp.dot(mask, h3, preferred_element_type=jnp.float32)
    gcnt = jnp.sum(mask, axis=1, keepdims=True)
    g = sums / jnp.maximum(gcnt, 1.0)
    g = _leaky(jnp.dot(g, w1_ref[...], preferred_element_type=jnp.float32)
               + b1_ref[...])
    g = _leaky(jnp.dot(g, w2_ref[...], preferred_element_type=jnp.float32)
               + b2_ref[...])
    o_ref[...] = (jnp.dot(g, w3_ref[...], preferred_element_type=jnp.float32)
                  + b3_ref[...])


def _finale(partial, counts, h, wu_top, wu_bot, bu, batch2d,
            w1, b1, w2, b2, w3, b3):
    rest = (h, wu_top, wu_bot, bu, batch2d, w1, b1, w2, b2, w3, b3)
    return pl.pallas_call(
        _finale_body,
        grid=(1,),
        in_specs=_PAD_SPECS + [_full_spec(x) for x in rest],
        out_specs=pl.BlockSpec((NUM_GRAPHS, 1), lambda i: (0, 0)),
        out_shape=jax.ShapeDtypeStruct((NUM_GRAPHS, 1), jnp.float32),
    )(partial, counts, *rest)


ECH = 64
IBK = 4
BLK = IBK * ECH
NBLK = N_EDGES // BLK
NITER = (NBLK + 2 * NW - 1) // (2 * NW)


def _sc_edge(apre, epre, src3d, dst3d, zeros_nd):
    mesh = plsc.VectorSubcoreMesh(core_axis_name="c", subcore_axis_name="s")

    def body(apre_hbm, epre_hbm, src_hbm, dst_hbm, z_hbm, out_hbm,
             gbuf0, ebuf0, seme0, semg0,
             gbuf1, ebuf1, seme1, semg1,
             sblk0, dblk0, sblk1, dblk1, acc):
        cid = jax.lax.axis_index("c")
        sid = jax.lax.axis_index("s")
        wid = sid * NC + cid
        row0 = pl.multiple_of(sid * ROWS_PT, 8)

        data = ((gbuf0, ebuf0, seme0, semg0), (gbuf1, ebuf1, seme1, semg1))
        iblk = ((sblk0, dblk0), (sblk1, dblk1))

        def load_iblk(b, ib):
            sblk, dblk = iblk[ib]
            pltpu.sync_copy(src_hbm.at[b], sblk)
            pltpu.sync_copy(dst_hbm.at[b], dblk)

        def start_chunk(b, c, ib, d):
            sblk, _ = iblk[ib]
            gbuf, ebuf, seme, semg = data[d]
            off = pl.multiple_of(b * BLK + c * ECH, 8)
            pltpu.make_async_copy(epre_hbm.at[pl.ds(off, ECH)], ebuf,
                                  seme).start()
            pltpu.make_async_copy(apre_hbm.at[sblk.at[c]], gbuf, semg).start()

        def finish_chunk(c, ib, d):
            sblk, dblk = iblk[ib]
            gbuf, ebuf, seme, semg = data[d]
            pltpu.make_async_copy(epre_hbm.at[pl.ds(0, ECH)], ebuf,
                                  seme).wait()
            pltpu.make_async_copy(apre_hbm.at[sblk.at[c]], gbuf, semg).wait()

            @pl.loop(0, ECH, step=4)
            def _row(i):
                for rr in range(4):
                    for j in range(D // LANES):
                        sl = pl.ds(j * LANES, LANES)
                        m = gbuf[i + rr, sl] + ebuf[i + rr, sl]
                        gbuf[i + rr, sl] = jnp.maximum(m, 0.2 * m)

            pltpu.sync_copy(gbuf, acc.at[dblk.at[c]], add=True)

        pltpu.sync_copy(z_hbm.at[sid], acc.at[pl.ds(row0, ROWS_PT)])
        plsc.subcore_barrier()

        load_iblk(wid, 0)
        start_chunk(wid, 0, 0, 0)

        @pl.loop(0, NITER)
        def _iter(t):
            b0 = wid + (2 * t) * NW
            b1 = b0 + NW
            bnext = b0 + 2 * NW
            gb1 = b1 < NBLK
            gnext = bnext < NBLK

            @pl.when(gb1)
            def _():
                load_iblk(b1, 1)

            start_chunk(b0, 1, 0, 1)
            finish_chunk(0, 0, 0)
            start_chunk(b0, 2, 0, 0)
            finish_chunk(1, 0, 1)
            start_chunk(b0, 3, 0, 1)
            finish_chunk(2, 0, 0)

            @pl.when(gb1)
            def _():
                start_chunk(b1, 0, 1, 0)

            finish_chunk(3, 0, 1)

            @pl.when(gb1)
            def _():
                start_chunk(b1, 1, 1, 1)
                finish_chunk(0, 1, 0)

                @pl.when(gnext)
                def _():
                    load_iblk(bnext, 0)

                start_chunk(b1, 2, 1, 0)
                finish_chunk(1, 1, 1)
                start_chunk(b1, 3, 1, 1)
                finish_chunk(2, 1, 0)

                @pl.when(gnext)
                def _():
                    start_chunk(bnext, 0, 0, 0)

                finish_chunk(3, 1, 1)

        plsc.subcore_barrier()
        pltpu.sync_copy(acc.at[pl.ds(row0, ROWS_PT)], out_hbm.at[cid, sid])

    k = pl.kernel(
        body,
        out_type=jax.ShapeDtypeStruct((NC, NS, ROWS_PT, D), jnp.float32),
        mesh=mesh,
        scratch_types=[
            pltpu.VMEM((ECH, D), jnp.float32),
            pltpu.VMEM((ECH, D), jnp.float32),
            pltpu.SemaphoreType.DMA,
            pltpu.SemaphoreType.DMA,
            pltpu.VMEM((ECH, D), jnp.float32),
            pltpu.VMEM((ECH, D), jnp.float32),
            pltpu.SemaphoreType.DMA,
            pltpu.SemaphoreType.DMA,
            pltpu.VMEM((IBK, ECH), jnp.int32),
            pltpu.VMEM((IBK, ECH), jnp.int32),
            pltpu.VMEM((IBK, ECH), jnp.int32),
            pltpu.VMEM((IBK, ECH), jnp.int32),
            pltpu.VMEM_SHARED((NROW_ACC, D), jnp.float32),
        ],
    )
    res = k(apre, epre, src3d, dst3d, zeros_nd)
    return res.reshape(NC, NROW_ACC, D)


def _sc_counts(dst, zeros_nd, ones_ebd):
    mesh = plsc.VectorSubcoreMesh(core_axis_name="c", subcore_axis_name="s")

    def body(dst_hbm, z_hbm, ones_hbm, cout_hbm, didx0, didx1, semi0, semi1,
             didxt, obuf, cacc):
        cid = jax.lax.axis_index("c")
        sid = jax.lax.axis_index("s")
        wid = sid * NC + cid
        row0 = pl.multiple_of(sid * ROWS_PT, 8)
        base = pl.multiple_of(wid * (N_EDGES // NW), 8)
        nch = (N_EDGES // NW) // EB

        ibufs = ((didx0, semi0), (didx1, semi1))

        def start_idx(ch, b):
            didx, semi = ibufs[b]
            off = pl.multiple_of(base + ch * EB, 8)
            pltpu.make_async_copy(dst_hbm.at[pl.ds(off, EB)], didx,
                                  semi).start()

        def finish(ch, b):
            didx, semi = ibufs[b]
            off = pl.multiple_of(base + ch * EB, 8)
            pltpu.make_async_copy(dst_hbm.at[pl.ds(off, EB)], didx,
                                  semi).wait()
            pltpu.sync_copy(obuf, cacc.at[didx], add=True)

        pltpu.sync_copy(z_hbm.at[sid], cacc.at[pl.ds(row0, ROWS_PT)])
        pltpu.sync_copy(ones_hbm, obuf)
        plsc.subcore_barrier()

        start_idx(0, 0)
        start_idx(1, 1)

        @pl.loop(0, nch // 2)
        def _pair(j):
            finish(2 * j, 0)

            @pl.when(j < nch // 2 - 1)
            def _():
                start_idx(2 * j + 2, 0)

            finish(2 * j + 1, 1)

            @pl.when(j < nch // 2 - 1)
            def _():
                start_idx(2 * j + 3, 1)

        offt = pl.multiple_of(base + nch * EB, 8)
        pltpu.sync_copy(dst_hbm.at[pl.ds(offt, 16)], didxt)
        pltpu.sync_copy(obuf.at[pl.ds(0, 16)], cacc.at[didxt], add=True)

        plsc.subcore_barrier()
        pltpu.sync_copy(cacc.at[pl.ds(row0, ROWS_PT)], cout_hbm.at[cid, sid])

    k = pl.kernel(
        body,
        out_type=jax.ShapeDtypeStruct((NC, NS, ROWS_PT, D), jnp.float32),
        mesh=mesh,
        scratch_types=[
            pltpu.VMEM((EB,), jnp.int32),
            pltpu.VMEM((EB,), jnp.int32),
            pltpu.SemaphoreType.DMA,
            pltpu.SemaphoreType.DMA,
            pltpu.VMEM((16,), jnp.int32),
            pltpu.VMEM((EB, D), jnp.float32),
            pltpu.VMEM_SHARED((NROW_ACC, D), jnp.float32),
        ],
    )
    res = k(dst, zeros_nd, ones_ebd)
    return res.reshape(NC, NROW_ACC, D)[:, :N_NODES, :LANES]


def kernel(x, edge_index, edge_attr, batch,
           Wm0, bm0, Wu0, bu0, Wm1, bm1, Wu1, bu1, Wm2, bm2, Wu2, bu2,
           W1, b1, W2, b2, W3, b3):
    src = edge_index[0]
    dst = edge_index[1]
    src3d = src.reshape(NBLK, IBK, ECH)
    dst3d = dst.reshape(NBLK, IBK, ECH)
    zeros_nd = jnp.zeros((NS, ROWS_PT, D), jnp.float32)
    ones_ebd = jnp.ones((EB, D), jnp.float32)

    cnts = _sc_counts(dst, zeros_nd, ones_ebd)
    ea_t = edge_attr.T
    e0 = _epre1(ea_t, Wm0[D:], bm0.reshape(1, D))
    apre0 = _apre(x, Wm0[:D])
    apre0, cnts = jax.lax.optimization_barrier((apre0, cnts))

    part0 = _sc_edge(apre0, e0, src3d, dst3d, zeros_nd)
    e1 = _epre1(ea_t, Wm1[D:], bm1.reshape(1, D))
    h1, apre1 = _update(part0, cnts, x, Wu0[:D], Wu0[D:],
                        bu0.reshape(1, D), Wm1[:D])
    part1 = _sc_edge(apre1, e1, src3d, dst3d, zeros_nd)
    e2 = _epre1(ea_t, Wm2[D:], bm2.reshape(1, D))
    h2, apre2 = _update(part1, cnts, h1, Wu1[:D], Wu1[D:],
                        bu1.reshape(1, D), Wm2[:D])
    part2 = _sc_edge(apre2, e2, src3d, dst3d, zeros_nd)

    return _finale(part2, cnts, h2, Wu2[:D], Wu2[D:], bu2.reshape(1, D),
                   batch.reshape(1, N_NODES), W1, b1.reshape(1, D),
                   W2, b2.reshape(1, 64), W3, b3.reshape(1, 1))
